# Initial kernel scaffold; baseline (speedup 1.0000x reference)
import jax
import jax.numpy as jnp
from jax.experimental import pallas as pl


def kernel(x, edge_index, edge_attr, W_node, W_edge, Wq, Wk, Wv):
    raise NotImplementedError("write your pallas kernel here")


# speedup vs baseline: 2.6842x; 2.6842x over previous
import dataclasses
import functools

import jax
import jax.numpy as jnp
from jax import lax
from jax.experimental import pallas as pl
from jax.experimental.pallas import tpu as pltpu
from jax.experimental.pallas import tpu_sc as plsc

N = 10000
E = 320000
DF = 128
DE = 16
D = 128
H = 4
DH = D // H
L = 16
NW = 32
C = 128
NCHUNKS = E // C
NCH_HI = -(-NCHUNKS // NW)
NP = 10240
NPH = NP * H
SLICE = NPH // 16
AGG_ROWS = NP // 16
INV_SQRT_DH = 1.0 / (DH ** 0.5)
NEG_INF = float("-inf")


def _mesh():
    return plsc.VectorSubcoreMesh(core_axis_name="c", subcore_axis_name="s")


def _sc_params():
    cp = pltpu.CompilerParams()
    if "needs_layout_passes" in pltpu.CompilerParams.__dataclass_fields__:
        cp = dataclasses.replace(cp, needs_layout_passes=False)
    return cp


def _worker_id():
    return lax.axis_index("s") * 2 + lax.axis_index("c")


def _vshuffle(x, idx):
    dn = lax.GatherDimensionNumbers(
        offset_dims=(), collapsed_slice_dims=(0,), start_index_map=(0,))
    return lax.gather(x, idx[:, None], dn, (1,),
                      mode=lax.GatherScatterMode.PROMISE_IN_BOUNDS)


def _dedup_combine(dstv, vals, is_max):
    lane = lax.iota(jnp.int32, L)
    sdst, perm = plsc.sort_key_val(dstv, lane)
    steps = []
    for d in (1, 2, 4, 8):
        nb = jnp.minimum(lane + d, L - 1)
        same = (_vshuffle(sdst, nb) == sdst) & (lane < L - d)
        steps.append((nb, same))
    prev = jnp.maximum(lane - 1, 0)
    firstm = (lane == 0) | (_vshuffle(sdst, prev) != sdst)
    ident = NEG_INF if is_max else 0.0
    comb = jnp.maximum if is_max else jnp.add
    out = []
    for v in vals:
        sv = _vshuffle(v, perm)
        for nb, same in steps:
            sv = comb(sv, jnp.where(same, _vshuffle(sv, nb), ident))
        out.append(sv)
    return sdst, firstm, out


def _foreach_chunk(wid, fn):
    @pl.loop(0, NCH_HI)
    def _(t):
        cidx = wid + NW * t

        @pl.when(cidx < NCHUNKS)
        def _():
            fn(cidx * C)


def _tc_tables(x, w_node, wq, wk, wv):
    blk = 1000

    def body(x_ref, wn_ref, wq_ref, wk_ref, wv_ref,
             xt_ref, q_ref, k_ref, v_ref):
        xt = jnp.dot(x_ref[...], wn_ref[...],
                     preferred_element_type=jnp.float32)
        xt_ref[...] = xt
        q_ref[...] = jnp.dot(xt, wq_ref[...],
                             preferred_element_type=jnp.float32)
        k_ref[...] = jnp.dot(xt, wk_ref[...],
                             preferred_element_type=jnp.float32)
        v_ref[...] = jnp.dot(xt, wv_ref[...],
                             preferred_element_type=jnp.float32)

    w_spec = pl.BlockSpec((DF, D), lambda i: (0, 0))
    r_spec = pl.BlockSpec((blk, D), lambda i: (i, 0))
    shp = jax.ShapeDtypeStruct((N, D), jnp.float32)
    return pl.pallas_call(
        body, grid=(N // blk,),
        in_specs=[pl.BlockSpec((blk, DF), lambda i: (i, 0)),
                  w_spec, w_spec, w_spec, w_spec],
        out_specs=[r_spec, r_spec, r_spec, r_spec],
        out_shape=[shp, shp, shp, shp],
    )(x, w_node, wq, wk, wv)


def _tc_edge_features(edge_attr, w_edge):
    blk = 4000

    def body(ea_ref, we_ref, ef_ref):
        ef_ref[...] = jnp.dot(ea_ref[...], we_ref[...],
                              preferred_element_type=jnp.float32)

    return pl.pallas_call(
        body, grid=(E // blk,),
        in_specs=[pl.BlockSpec((blk, DE), lambda i: (i, 0)),
                  pl.BlockSpec((DE, D), lambda i: (0, 0))],
        out_specs=pl.BlockSpec((blk, D), lambda i: (i, 0)),
        out_shape=jax.ShapeDtypeStruct((E, D), jnp.float32),
    )(edge_attr, w_edge)


def _tc_final(xt, agg_part, den2d):
    blk = 1000

    def body(xt_ref, ag_ref, dn_ref, o_ref):
        den128 = jnp.repeat(dn_ref[...], DH, axis=1)
        agg = (ag_ref[0] + ag_ref[1]) / (den128 + 1e-9)
        o_ref[...] = jnp.maximum(xt_ref[...] + agg, 0.0)

    return pl.pallas_call(
        body, grid=(N // blk,),
        in_specs=[pl.BlockSpec((blk, D), lambda i: (i, 0)),
                  pl.BlockSpec((2, blk, D), lambda i: (0, i, 0)),
                  pl.BlockSpec((blk, H), lambda i: (i, 0))],
        out_specs=pl.BlockSpec((blk, D), lambda i: (i, 0)),
        out_shape=jax.ShapeDtypeStruct((N, D), jnp.float32),
    )(xt, agg_part, den2d)


def _sc_logits(q, k, ef, src, dst):
    @functools.partial(
        pl.kernel,
        out_type=jax.ShapeDtypeStruct((H, E), jnp.float32),
        mesh=_mesh(),
        compiler_params=_sc_params(),
        scratch_types=[
            pltpu.VMEM((1, C), jnp.int32),
            pltpu.VMEM((1, C), jnp.int32),
            pltpu.VMEM((C, D), jnp.float32),
            pltpu.VMEM((C, D), jnp.float32),
            pltpu.VMEM((C, D), jnp.float32),
            pltpu.VMEM((H, C), jnp.float32),
        ],
    )
    def kern(q_hbm, k_hbm, ef_hbm, src_hbm, dst_hbm, lo_hbm,
             di, si, qb, kb, eb, lb):
        wid = _worker_id()
        lane = lax.iota(jnp.int32, L)
        lastm = lane == (L - 1)

        def chunk(off):
            pltpu.sync_copy(dst_hbm.at[pl.ds(off, C)], di.at[0])
            pltpu.sync_copy(src_hbm.at[pl.ds(off, C)], si.at[0])
            pltpu.sync_copy(q_hbm.at[di.at[0]], qb)
            pltpu.sync_copy(k_hbm.at[si.at[0]], kb)
            pltpu.sync_copy(ef_hbm.at[pl.ds(off, C)], eb)

            @pl.loop(0, C)
            def _(e):
                he = lane * 0 + e
                for h in range(H):
                    j0, j1 = 2 * h, 2 * h + 1
                    t = (qb[e, pl.ds(L * j0, L)]
                         * (kb[e, pl.ds(L * j0, L)] + eb[e, pl.ds(L * j0, L)])
                         + qb[e, pl.ds(L * j1, L)]
                         * (kb[e, pl.ds(L * j1, L)] + eb[e, pl.ds(L * j1, L)]))
                    cs = plsc.cumsum(t) * INV_SQRT_DH
                    plsc.store_scatter(lb, [lane * 0 + h, he], cs,
                                       mask=lastm)

            pltpu.sync_copy(lb, lo_hbm.at[:, pl.ds(off, C)])

        _foreach_chunk(wid, chunk)

    return kern(q, k, ef, src, dst)


def _sc_segmax(lo, dst):
    @functools.partial(
        pl.kernel,
        out_type=jax.ShapeDtypeStruct((NW * NPH,), jnp.float32),
        mesh=_mesh(),
        compiler_params=_sc_params(),
        scratch_types=[
            pltpu.VMEM((NPH,), jnp.float32),
            pltpu.VMEM((H, C), jnp.float32),
            pltpu.VMEM((1, C), jnp.int32),
        ],
    )
    def kern(lo_hbm, dst_hbm, mp_hbm, m_loc, lb, di):
        wid = _worker_id()

        @pl.loop(0, NPH, step=L)
        def _(i):
            m_loc[pl.ds(i, L)] = jnp.full((L,), NEG_INF, jnp.float32)

        def chunk(off):
            pltpu.sync_copy(dst_hbm.at[pl.ds(off, C)], di.at[0])
            pltpu.sync_copy(lo_hbm.at[:, pl.ds(off, C)], lb)
            for g in range(C // L):
                dstv = di[0, pl.ds(L * g, L)]
                vals = [lb[h, pl.ds(L * g, L)] for h in range(H)]
                sdst, firstm, svals = _dedup_combine(dstv, vals, True)
                b4 = sdst * H
                for h in range(H):
                    idx = b4 + h
                    cur = plsc.load_gather(m_loc, [idx])
                    plsc.store_scatter(m_loc, [idx],
                                       jnp.maximum(cur, svals[h]),
                                       mask=firstm)

        _foreach_chunk(wid, chunk)
        pltpu.sync_copy(m_loc, mp_hbm.at[pl.ds(wid * NPH, NPH)])

    return kern(lo, dst)


def _sc_combine(parts, is_max):
    cw = NPH // NW

    @functools.partial(
        pl.kernel,
        out_type=jax.ShapeDtypeStruct((NPH,), jnp.float32),
        mesh=_mesh(),
        compiler_params=_sc_params(),
        scratch_types=[
            pltpu.VMEM((cw,), jnp.float32),
            pltpu.VMEM((cw,), jnp.float32),
        ],
    )
    def kern(parts_hbm, out_hbm, acc, tmp):
        wid = _worker_id()
        off = wid * cw
        comb = jnp.maximum if is_max else jnp.add
        pltpu.sync_copy(parts_hbm.at[pl.ds(off, cw)], acc)
        for t in range(1, NW):
            pltpu.sync_copy(parts_hbm.at[pl.ds(t * NPH + off, cw)], tmp)

            @pl.loop(0, cw, step=L)
            def _(i):
                acc[pl.ds(i, L)] = comb(acc[pl.ds(i, L)], tmp[pl.ds(i, L)])

        pltpu.sync_copy(acc, out_hbm.at[pl.ds(off, cw)])

    return kern(parts)


def _sc_exp_den(lo, dst, m_final):
    @functools.partial(
        pl.kernel,
        out_type=(jax.ShapeDtypeStruct((H, E), jnp.float32),
                  jax.ShapeDtypeStruct((NW * NPH,), jnp.float32)),
        mesh=_mesh(),
        compiler_params=_sc_params(),
        scratch_types=[
            pltpu.VMEM((NPH,), jnp.float32),
            pltpu.VMEM((NPH,), jnp.float32),
            pltpu.VMEM((H, C), jnp.float32),
            pltpu.VMEM((H, C), jnp.float32),
            pltpu.VMEM((1, C), jnp.int32),
        ],
    )
    def kern(lo_hbm, dst_hbm, m_hbm, ex_hbm, dp_hbm,
             m_loc, den_loc, lb, exb, di):
        wid = _worker_id()
        pltpu.sync_copy(m_hbm, m_loc)

        @pl.loop(0, NPH, step=L)
        def _(i):
            den_loc[pl.ds(i, L)] = jnp.zeros((L,), jnp.float32)

        def chunk(off):
            pltpu.sync_copy(dst_hbm.at[pl.ds(off, C)], di.at[0])
            pltpu.sync_copy(lo_hbm.at[:, pl.ds(off, C)], lb)
            for g in range(C // L):
                dstv = di[0, pl.ds(L * g, L)]
                b4 = dstv * H
                exs = []
                for h in range(H):
                    mg = plsc.load_gather(m_loc, [b4 + h])
                    exv = jnp.exp(lb[h, pl.ds(L * g, L)] - mg)
                    exb[h, pl.ds(L * g, L)] = exv
                    exs.append(exv)
                sdst, firstm, svals = _dedup_combine(dstv, exs, False)
                sb4 = sdst * H
                for h in range(H):
                    idx = sb4 + h
                    cur = plsc.load_gather(den_loc, [idx])
                    plsc.store_scatter(den_loc, [idx], cur + svals[h],
                                       mask=firstm)
            pltpu.sync_copy(exb, ex_hbm.at[:, pl.ds(off, C)])

        _foreach_chunk(wid, chunk)
        pltpu.sync_copy(den_loc, dp_hbm.at[pl.ds(wid * NPH, NPH)])

    return kern(lo, dst, m_final)


def _sc_aggregate(v, ef, ex, src, dst):
    @functools.partial(
        pl.kernel,
        out_type=jax.ShapeDtypeStruct((2, N, D), jnp.float32),
        mesh=_mesh(),
        compiler_params=_sc_params(),
        scratch_types=[
            pltpu.VMEM((1, C), jnp.int32),
            pltpu.VMEM((1, C), jnp.int32),
            pltpu.VMEM((C, D), jnp.float32),
            pltpu.VMEM((C, D), jnp.float32),
            pltpu.VMEM((H, C), jnp.float32),
            pltpu.VMEM_SHARED((NP, D), jnp.float32),
        ],
    )
    def kern(v_hbm, ef_hbm, ex_hbm, src_hbm, dst_hbm, ag_hbm,
             di, si, vb, eb, exb, agg_sp):
        cid = lax.axis_index("c")
        sid = lax.axis_index("s")
        wid = _worker_id()
        lane = lax.iota(jnp.int32, L)

        @pl.loop(0, C)
        def _(r):
            @pl.loop(0, D, step=L)
            def _(c0):
                vb[r, pl.ds(c0, L)] = jnp.zeros((L,), jnp.float32)

        row0 = sid * AGG_ROWS
        for b in range(AGG_ROWS // C):
            pltpu.sync_copy(vb, agg_sp.at[pl.ds(row0 + b * C, C)])
        plsc.subcore_barrier()

        def chunk(off):
            pltpu.sync_copy(dst_hbm.at[pl.ds(off, C)], di.at[0])
            pltpu.sync_copy(src_hbm.at[pl.ds(off, C)], si.at[0])
            pltpu.sync_copy(v_hbm.at[si.at[0]], vb)
            pltpu.sync_copy(ef_hbm.at[pl.ds(off, C)], eb)
            pltpu.sync_copy(ex_hbm.at[:, pl.ds(off, C)], exb)
            for g in range(C // L):
                exs = [exb[h, pl.ds(L * g, L)] for h in range(H)]

                @pl.loop(0, L)
                def _(e16):
                    row = L * g + e16
                    sel = lane * 0 + e16
                    bc = [_vshuffle(exs[h], sel) for h in range(H)]
                    for j in range(D // L):
                        vb[row, pl.ds(L * j, L)] = (
                            (vb[row, pl.ds(L * j, L)]
                             + eb[row, pl.ds(L * j, L)]) * bc[j // 2])

            pltpu.sync_copy(vb, agg_sp.at[di.at[0]], add=True)

        _foreach_chunk(wid, chunk)

        plsc.subcore_barrier()
        nvalid = N - 15 * AGG_ROWS

        @pl.when(sid < 15)
        def _():
            for b in range(AGG_ROWS // C):
                pltpu.sync_copy(agg_sp.at[pl.ds(row0 + b * C, C)],
                                ag_hbm.at[cid, pl.ds(row0 + b * C, C)])

        @pl.when(sid == 15)
        def _():
            for b in range(nvalid // C):
                pltpu.sync_copy(agg_sp.at[pl.ds(row0 + b * C, C)],
                                ag_hbm.at[cid, pl.ds(row0 + b * C, C)])
            rem = nvalid - (nvalid // C) * C
            if rem:
                pltpu.sync_copy(
                    agg_sp.at[pl.ds(row0 + (nvalid // C) * C, rem)],
                    ag_hbm.at[cid, pl.ds(row0 + (nvalid // C) * C, rem)])

    return kern(v, ef, ex, src, dst)


def kernel(x, edge_index, edge_attr, W_node, W_edge, Wq, Wk, Wv):
    src = edge_index[0]
    dst = edge_index[1]
    xt, q, k, v = _tc_tables(x, W_node, Wq, Wk, Wv)
    ef = _tc_edge_features(edge_attr, W_edge)
    lo = _sc_logits(q, k, ef, src, dst)
    m_parts = _sc_segmax(lo, dst)
    m_final = _sc_combine(m_parts, True)
    ex, den_parts = _sc_exp_den(lo, dst, m_final)
    den_final = _sc_combine(den_parts, False)
    agg_part = _sc_aggregate(v, ef, ex, src, dst)
    den2d = den_final[:N * H].reshape(N, H)
    return _tc_final(xt, agg_part, den2d)


# speedup vs baseline: 3.1892x; 1.1881x over previous
import dataclasses
import functools

import jax
import jax.numpy as jnp
from jax import lax
from jax.experimental import pallas as pl
from jax.experimental.pallas import tpu as pltpu
from jax.experimental.pallas import tpu_sc as plsc

N = 10000
E = 320000
DF = 128
DE = 16
D = 128
H = 4
DH = D // H
L = 16
NW = 32
C = 128
NCHUNKS = E // C
NCH_HI = -(-NCHUNKS // NW)
NP = 10240
NPH = NP * H
SLICE = NPH // 16
AGG_ROWS = NP // 16
INV_SQRT_DH = 1.0 / (DH ** 0.5)
NEG_INF = float("-inf")


def _mesh():
    return plsc.VectorSubcoreMesh(core_axis_name="c", subcore_axis_name="s")


def _sc_params():
    cp = pltpu.CompilerParams()
    if "needs_layout_passes" in pltpu.CompilerParams.__dataclass_fields__:
        cp = dataclasses.replace(cp, needs_layout_passes=False)
    return cp


def _worker_id():
    return lax.axis_index("s") * 2 + lax.axis_index("c")


def _vshuffle(x, idx):
    dn = lax.GatherDimensionNumbers(
        offset_dims=(), collapsed_slice_dims=(0,), start_index_map=(0,))
    return lax.gather(x, idx[:, None], dn, (1,),
                      mode=lax.GatherScatterMode.PROMISE_IN_BOUNDS)


def _dedup_combine(dstv, vals, is_max):
    lane = lax.iota(jnp.int32, L)
    sdst, perm = plsc.sort_key_val(dstv, lane)
    steps = []
    for d in (1, 2, 4, 8):
        nb = jnp.minimum(lane + d, L - 1)
        same = (_vshuffle(sdst, nb) == sdst) & (lane < L - d)
        steps.append((nb, same))
    prev = jnp.maximum(lane - 1, 0)
    firstm = (lane == 0) | (_vshuffle(sdst, prev) != sdst)
    ident = NEG_INF if is_max else 0.0
    comb = jnp.maximum if is_max else jnp.add
    out = []
    for v in vals:
        sv = _vshuffle(v, perm)
        for nb, same in steps:
            sv = comb(sv, jnp.where(same, _vshuffle(sv, nb), ident))
        out.append(sv)
    return sdst, firstm, out


def _foreach_chunk(wid, fn):
    @pl.loop(0, NCH_HI)
    def _(t):
        cidx = wid + NW * t

        @pl.when(cidx < NCHUNKS)
        def _():
            fn(cidx * C)


def _tc_tables(x, w_node, wq, wk, wv):
    blk = 1000

    def body(x_ref, wn_ref, wq_ref, wk_ref, wv_ref,
             xt_ref, q_ref, k_ref, v_ref):
        xt = jnp.dot(x_ref[...], wn_ref[...],
                     preferred_element_type=jnp.float32)
        xt_ref[...] = xt
        q_ref[...] = jnp.dot(xt, wq_ref[...],
                             preferred_element_type=jnp.float32)
        k_ref[...] = jnp.dot(xt, wk_ref[...],
                             preferred_element_type=jnp.float32)
        v_ref[...] = jnp.dot(xt, wv_ref[...],
                             preferred_element_type=jnp.float32)

    w_spec = pl.BlockSpec((DF, D), lambda i: (0, 0))
    r_spec = pl.BlockSpec((blk, D), lambda i: (i, 0))
    shp = jax.ShapeDtypeStruct((N, D), jnp.float32)
    return pl.pallas_call(
        body, grid=(N // blk,),
        in_specs=[pl.BlockSpec((blk, DF), lambda i: (i, 0)),
                  w_spec, w_spec, w_spec, w_spec],
        out_specs=[r_spec, r_spec, r_spec, r_spec],
        out_shape=[shp, shp, shp, shp],
    )(x, w_node, wq, wk, wv)


def _tc_edge_features(edge_attr, w_edge):
    blk = 4000

    def body(ea_ref, we_ref, ef_ref):
        ef_ref[...] = jnp.dot(ea_ref[...], we_ref[...],
                              preferred_element_type=jnp.float32)

    return pl.pallas_call(
        body, grid=(E // blk,),
        in_specs=[pl.BlockSpec((blk, DE), lambda i: (i, 0)),
                  pl.BlockSpec((DE, D), lambda i: (0, 0))],
        out_specs=pl.BlockSpec((blk, D), lambda i: (i, 0)),
        out_shape=jax.ShapeDtypeStruct((E, D), jnp.float32),
    )(edge_attr, w_edge)


def _tc_final(xt, agg_part, den2d):
    blk = 1000

    def body(xt_ref, ag_ref, dn_ref, o_ref):
        den128 = jnp.repeat(dn_ref[...], DH, axis=1)
        agg = (ag_ref[0] + ag_ref[1]) / (den128 + 1e-9)
        o_ref[...] = jnp.maximum(xt_ref[...] + agg, 0.0)

    return pl.pallas_call(
        body, grid=(N // blk,),
        in_specs=[pl.BlockSpec((blk, D), lambda i: (i, 0)),
                  pl.BlockSpec((2, blk, D), lambda i: (0, i, 0)),
                  pl.BlockSpec((blk, H), lambda i: (i, 0))],
        out_specs=pl.BlockSpec((blk, D), lambda i: (i, 0)),
        out_shape=jax.ShapeDtypeStruct((N, D), jnp.float32),
    )(xt, agg_part, den2d)


def _sc_logits(q, k, ef, src, dst):
    @functools.partial(
        pl.kernel,
        out_type=jax.ShapeDtypeStruct((H, E), jnp.float32),
        mesh=_mesh(),
        compiler_params=_sc_params(),
        scratch_types=[
            pltpu.VMEM((1, C), jnp.int32),
            pltpu.VMEM((1, C), jnp.int32),
            pltpu.VMEM((1, C), jnp.int32),
            pltpu.VMEM((1, C), jnp.int32),
            pltpu.VMEM((C, D), jnp.float32),
            pltpu.VMEM((C, D), jnp.float32),
            pltpu.VMEM((C, D), jnp.float32),
            pltpu.VMEM((C, D), jnp.float32),
            pltpu.VMEM((C, D), jnp.float32),
            pltpu.VMEM((C, D), jnp.float32),
            pltpu.VMEM((H, C), jnp.float32),
            pltpu.SemaphoreType.DMA,
            pltpu.SemaphoreType.DMA,
        ],
    )
    def kern(q_hbm, k_hbm, ef_hbm, src_hbm, dst_hbm, lo_hbm,
             di0, di1, si0, si1, qb0, qb1, kb0, kb1, eb0, eb1, lb,
             sem0, sem1):
        wid = _worker_id()
        nmy = jnp.where(wid < NCHUNKS - (NCH_HI - 1) * NW,
                        NCH_HI, NCH_HI - 1)
        lane = lax.iota(jnp.int32, L)
        lastm = lane == (L - 1)
        slots = ((di0, si0, qb0, kb0, eb0, sem0),
                 (di1, si1, qb1, kb1, eb1, sem1))

        def issue(s, t):
            di, si, qb, kb, eb, sem = slots[s]
            off = (wid + NW * t) * C
            pltpu.sync_copy(dst_hbm.at[pl.ds(off, C)], di.at[0])
            pltpu.sync_copy(src_hbm.at[pl.ds(off, C)], si.at[0])
            pltpu.async_copy(q_hbm.at[di.at[0]], qb, sem)
            pltpu.async_copy(k_hbm.at[si.at[0]], kb, sem)
            pltpu.async_copy(ef_hbm.at[pl.ds(off, C)], eb, sem)

        def consume(s, t):
            di, si, qb, kb, eb, sem = slots[s]
            off = (wid + NW * t) * C
            pltpu.make_async_copy(q_hbm.at[di.at[0]], qb, sem).wait()
            pltpu.make_async_copy(k_hbm.at[si.at[0]], kb, sem).wait()
            pltpu.make_async_copy(ef_hbm.at[pl.ds(off, C)], eb, sem).wait()

            @pl.loop(0, C)
            def _(e):
                he = lane * 0 + e
                for h in range(H):
                    j0, j1 = 2 * h, 2 * h + 1
                    t_ = (qb[e, pl.ds(L * j0, L)]
                          * (kb[e, pl.ds(L * j0, L)]
                             + eb[e, pl.ds(L * j0, L)])
                          + qb[e, pl.ds(L * j1, L)]
                          * (kb[e, pl.ds(L * j1, L)]
                             + eb[e, pl.ds(L * j1, L)]))
                    cs = plsc.cumsum(t_) * INV_SQRT_DH
                    plsc.store_scatter(lb, [lane * 0 + h, he], cs,
                                       mask=lastm)

            pltpu.sync_copy(lb, lo_hbm.at[:, pl.ds(off, C)])

        @pl.when(0 < nmy)
        def _():
            issue(0, 0)

        @pl.loop(0, (NCH_HI + 1) // 2)
        def _(i):
            t0 = 2 * i
            t1 = 2 * i + 1

            @pl.when(t1 < nmy)
            def _():
                issue(1, t1)

            @pl.when(t0 < nmy)
            def _():
                consume(0, t0)

            @pl.when(t1 + 1 < nmy)
            def _():
                issue(0, t1 + 1)

            @pl.when(t1 < nmy)
            def _():
                consume(1, t1)

    return kern(q, k, ef, src, dst)


def _sc_segmax(lo, dst):
    @functools.partial(
        pl.kernel,
        out_type=jax.ShapeDtypeStruct((NW * NPH,), jnp.float32),
        mesh=_mesh(),
        compiler_params=_sc_params(),
        scratch_types=[
            pltpu.VMEM((NPH,), jnp.float32),
            pltpu.VMEM((H, C), jnp.float32),
            pltpu.VMEM((1, C), jnp.int32),
        ],
    )
    def kern(lo_hbm, dst_hbm, mp_hbm, m_loc, lb, di):
        wid = _worker_id()

        @pl.loop(0, NPH, step=L)
        def _(i):
            m_loc[pl.ds(i, L)] = jnp.full((L,), NEG_INF, jnp.float32)

        def chunk(off):
            pltpu.sync_copy(dst_hbm.at[pl.ds(off, C)], di.at[0])
            pltpu.sync_copy(lo_hbm.at[:, pl.ds(off, C)], lb)
            for g in range(C // L):
                dstv = di[0, pl.ds(L * g, L)]
                vals = [lb[h, pl.ds(L * g, L)] for h in range(H)]
                sdst, firstm, svals = _dedup_combine(dstv, vals, True)
                b4 = sdst * H
                for h in range(H):
                    idx = b4 + h
                    cur = plsc.load_gather(m_loc, [idx])
                    plsc.store_scatter(m_loc, [idx],
                                       jnp.maximum(cur, svals[h]),
                                       mask=firstm)

        _foreach_chunk(wid, chunk)
        pltpu.sync_copy(m_loc, mp_hbm.at[pl.ds(wid * NPH, NPH)])

    return kern(lo, dst)


def _sc_combine(parts, is_max):
    cw = NPH // NW

    @functools.partial(
        pl.kernel,
        out_type=jax.ShapeDtypeStruct((NPH,), jnp.float32),
        mesh=_mesh(),
        compiler_params=_sc_params(),
        scratch_types=[
            pltpu.VMEM((cw,), jnp.float32),
            pltpu.VMEM((cw,), jnp.float32),
        ],
    )
    def kern(parts_hbm, out_hbm, acc, tmp):
        wid = _worker_id()
        off = wid * cw
        comb = jnp.maximum if is_max else jnp.add
        pltpu.sync_copy(parts_hbm.at[pl.ds(off, cw)], acc)
        for t in range(1, NW):
            pltpu.sync_copy(parts_hbm.at[pl.ds(t * NPH + off, cw)], tmp)

            @pl.loop(0, cw, step=L)
            def _(i):
                acc[pl.ds(i, L)] = comb(acc[pl.ds(i, L)], tmp[pl.ds(i, L)])

        pltpu.sync_copy(acc, out_hbm.at[pl.ds(off, cw)])

    return kern(parts)


def _sc_exp_den(lo, dst, m_final):
    @functools.partial(
        pl.kernel,
        out_type=(jax.ShapeDtypeStruct((E, H), jnp.float32),
                  jax.ShapeDtypeStruct((NW * NPH,), jnp.float32)),
        mesh=_mesh(),
        compiler_params=_sc_params(),
        scratch_types=[
            pltpu.VMEM((NPH,), jnp.float32),
            pltpu.VMEM((NPH,), jnp.float32),
            pltpu.VMEM((H, C), jnp.float32),
            pltpu.VMEM((C, H), jnp.float32),
            pltpu.VMEM((1, C), jnp.int32),
        ],
    )
    def kern(lo_hbm, dst_hbm, m_hbm, ex_hbm, dp_hbm,
             m_loc, den_loc, lb, exb, di):
        lane = lax.iota(jnp.int32, L)
        wid = _worker_id()
        pltpu.sync_copy(m_hbm, m_loc)

        @pl.loop(0, NPH, step=L)
        def _(i):
            den_loc[pl.ds(i, L)] = jnp.zeros((L,), jnp.float32)

        def chunk(off):
            pltpu.sync_copy(dst_hbm.at[pl.ds(off, C)], di.at[0])
            pltpu.sync_copy(lo_hbm.at[:, pl.ds(off, C)], lb)
            for g in range(C // L):
                dstv = di[0, pl.ds(L * g, L)]
                b4 = dstv * H
                exs = []
                erow = lane + L * g
                for h in range(H):
                    mg = plsc.load_gather(m_loc, [b4 + h])
                    exv = jnp.exp(lb[h, pl.ds(L * g, L)] - mg)
                    plsc.store_scatter(exb, [erow, lane * 0 + h], exv)
                    exs.append(exv)
                sdst, firstm, svals = _dedup_combine(dstv, exs, False)
                sb4 = sdst * H
                for h in range(H):
                    idx = sb4 + h
                    cur = plsc.load_gather(den_loc, [idx])
                    plsc.store_scatter(den_loc, [idx], cur + svals[h],
                                       mask=firstm)
            pltpu.sync_copy(exb, ex_hbm.at[pl.ds(off, C)])

        _foreach_chunk(wid, chunk)
        pltpu.sync_copy(den_loc, dp_hbm.at[pl.ds(wid * NPH, NPH)])

    return kern(lo, dst, m_final)


def _sc_aggregate(v, ef, ex, src, dst):
    C4 = 64
    nch4 = E // C4
    hi4 = -(-nch4 // NW)

    @functools.partial(
        pl.kernel,
        out_type=jax.ShapeDtypeStruct((2, N, D), jnp.float32),
        mesh=_mesh(),
        compiler_params=_sc_params(),
        scratch_types=[
            pltpu.VMEM((1, C4), jnp.int32),
            pltpu.VMEM((1, C4), jnp.int32),
            pltpu.VMEM((1, C4), jnp.int32),
            pltpu.VMEM((1, C4), jnp.int32),
            pltpu.VMEM((C4, D), jnp.float32),
            pltpu.VMEM((C4, D), jnp.float32),
            pltpu.VMEM((C4, D), jnp.float32),
            pltpu.VMEM((C4, D), jnp.float32),
            pltpu.VMEM((C4, H), jnp.float32),
            pltpu.VMEM((C4, H), jnp.float32),
            pltpu.SemaphoreType.DMA,
            pltpu.SemaphoreType.DMA,
            pltpu.VMEM_SHARED((N, D), jnp.float32),
        ],
    )
    def kern(v_hbm, ef_hbm, ex_hbm, src_hbm, dst_hbm, ag_hbm,
             di0, di1, si0, si1, vb0, vb1, eb0, eb1, xb0, xb1,
             sem0, sem1, agg_sp):
        cid = lax.axis_index("c")
        sid = lax.axis_index("s")
        wid = _worker_id()
        lane = lax.iota(jnp.int32, L)
        nmy = jnp.where(wid < nch4 - (hi4 - 1) * NW, hi4, hi4 - 1)
        slots = ((di0, si0, vb0, eb0, xb0, sem0),
                 (di1, si1, vb1, eb1, xb1, sem1))

        @pl.loop(0, C4)
        def _(r):
            @pl.loop(0, D, step=L)
            def _(c0):
                vb0[r, pl.ds(c0, L)] = jnp.zeros((L,), jnp.float32)

        RT = 624
        row0 = sid * RT

        @pl.when(sid < 15)
        def _():
            for b in range(RT // C4):
                pltpu.sync_copy(vb0, agg_sp.at[pl.ds(row0 + b * C4, C4)])
            pltpu.sync_copy(vb0.at[pl.ds(0, RT - (RT // C4) * C4)],
                            agg_sp.at[pl.ds(row0 + (RT // C4) * C4,
                                            RT - (RT // C4) * C4)])

        @pl.when(sid == 15)
        def _():
            for b in range((N - 15 * RT) // C4):
                pltpu.sync_copy(vb0, agg_sp.at[pl.ds(row0 + b * C4, C4)])

        plsc.subcore_barrier()

        def issue(s, t):
            di, si, vb, eb, xb, sem = slots[s]
            off = (wid + NW * t) * C4
            pltpu.sync_copy(dst_hbm.at[pl.ds(off, C4)], di.at[0])
            pltpu.sync_copy(src_hbm.at[pl.ds(off, C4)], si.at[0])
            pltpu.async_copy(v_hbm.at[si.at[0]], vb, sem)
            pltpu.async_copy(ef_hbm.at[pl.ds(off, C4)], eb, sem)
            pltpu.async_copy(ex_hbm.at[pl.ds(off, C4)], xb, sem)

        def consume(s, t):
            di, si, vb, eb, xb, sem = slots[s]
            off = (wid + NW * t) * C4
            pltpu.make_async_copy(v_hbm.at[si.at[0]], vb, sem).wait()
            pltpu.make_async_copy(ef_hbm.at[pl.ds(off, C4)], eb, sem).wait()
            pltpu.make_async_copy(ex_hbm.at[pl.ds(off, C4)], xb, sem).wait()
            for g in range(C4 // L):
                erow = lane + L * g
                exs = [plsc.load_gather(xb, [erow, lane * 0 + h])
                       for h in range(H)]

                @pl.loop(0, L)
                def _(e16):
                    row = L * g + e16
                    sel = lane * 0 + e16
                    bc = [_vshuffle(exs[h], sel) for h in range(H)]
                    for j in range(D // L):
                        vb[row, pl.ds(L * j, L)] = (
                            (vb[row, pl.ds(L * j, L)]
                             + eb[row, pl.ds(L * j, L)]) * bc[j // 2])

            pltpu.sync_copy(vb, agg_sp.at[di.at[0]], add=True)

        @pl.when(0 < nmy)
        def _():
            issue(0, 0)

        @pl.loop(0, (hi4 + 1) // 2)
        def _(i):
            t0 = 2 * i
            t1 = 2 * i + 1

            @pl.when(t1 < nmy)
            def _():
                issue(1, t1)

            @pl.when(t0 < nmy)
            def _():
                consume(0, t0)

            @pl.when(t1 + 1 < nmy)
            def _():
                issue(0, t1 + 1)

            @pl.when(t1 < nmy)
            def _():
                consume(1, t1)

        plsc.subcore_barrier()

        @pl.when(sid < 15)
        def _():
            for b in range(RT // C):
                pltpu.sync_copy(agg_sp.at[pl.ds(row0 + b * C, C)],
                                ag_hbm.at[cid, pl.ds(row0 + b * C, C)])
            rem = RT - (RT // C) * C
            pltpu.sync_copy(
                agg_sp.at[pl.ds(row0 + (RT // C) * C, rem)],
                ag_hbm.at[cid, pl.ds(row0 + (RT // C) * C, rem)])

        @pl.when(sid == 15)
        def _():
            nv = N - 15 * RT
            for b in range(nv // C):
                pltpu.sync_copy(agg_sp.at[pl.ds(row0 + b * C, C)],
                                ag_hbm.at[cid, pl.ds(row0 + b * C, C)])

    return kern(v, ef, ex, src, dst)


def kernel(x, edge_index, edge_attr, W_node, W_edge, Wq, Wk, Wv):
    src = edge_index[0]
    dst = edge_index[1]
    xt, q, k, v = _tc_tables(x, W_node, Wq, Wk, Wv)
    ef = _tc_edge_features(edge_attr, W_edge)
    lo = _sc_logits(q, k, ef, src, dst)
    m_parts = _sc_segmax(lo, dst)
    m_final = _sc_combine(m_parts, True)
    ex, den_parts = _sc_exp_den(lo, dst, m_final)
    den_final = _sc_combine(den_parts, False)
    agg_part = _sc_aggregate(v, ef, ex, src, dst)
    den2d = den_final[:N * H].reshape(N, H)
    return _tc_final(xt, agg_part, den2d)


# speedup vs baseline: 3.5435x; 1.1111x over previous
import dataclasses
import functools

import jax
import jax.numpy as jnp
from jax import lax
from jax.experimental import pallas as pl
from jax.experimental.pallas import tpu as pltpu
from jax.experimental.pallas import tpu_sc as plsc

N = 10000
E = 320000
DF = 128
DE = 16
D = 128
H = 4
DH = D // H
L = 16
NW = 32
C = 128
NCHUNKS = E // C
NCH_HI = -(-NCHUNKS // NW)
NP = 10240
NPH = NP * H
SLICE = NPH // 16
AGG_ROWS = NP // 16
INV_SQRT_DH = 1.0 / (DH ** 0.5)
NEG_INF = float("-inf")


def _mesh():
    return plsc.VectorSubcoreMesh(core_axis_name="c", subcore_axis_name="s")


def _sc_params():
    cp = pltpu.CompilerParams()
    if "needs_layout_passes" in pltpu.CompilerParams.__dataclass_fields__:
        cp = dataclasses.replace(cp, needs_layout_passes=False)
    return cp


def _worker_id():
    return lax.axis_index("s") * 2 + lax.axis_index("c")


def _vshuffle(x, idx):
    dn = lax.GatherDimensionNumbers(
        offset_dims=(), collapsed_slice_dims=(0,), start_index_map=(0,))
    return lax.gather(x, idx[:, None], dn, (1,),
                      mode=lax.GatherScatterMode.PROMISE_IN_BOUNDS)


def _dedup_combine(dstv, vals, is_max):
    lane = lax.iota(jnp.int32, L)
    sdst, perm = plsc.sort_key_val(dstv, lane)
    steps = []
    for d in (1, 2, 4, 8):
        nb = jnp.minimum(lane + d, L - 1)
        same = (_vshuffle(sdst, nb) == sdst) & (lane < L - d)
        steps.append((nb, same))
    prev = jnp.maximum(lane - 1, 0)
    firstm = (lane == 0) | (_vshuffle(sdst, prev) != sdst)
    ident = NEG_INF if is_max else 0.0
    comb = jnp.maximum if is_max else jnp.add
    out = []
    for v in vals:
        sv = _vshuffle(v, perm)
        for nb, same in steps:
            sv = comb(sv, jnp.where(same, _vshuffle(sv, nb), ident))
        out.append(sv)
    return sdst, firstm, out


def _foreach_chunk(wid, fn):
    @pl.loop(0, NCH_HI)
    def _(t):
        cidx = wid + NW * t

        @pl.when(cidx < NCHUNKS)
        def _():
            fn(cidx * C)


def _tc_tables(x, w_node, wq, wk, wv):
    blk = 1000

    def body(x_ref, wn_ref, wq_ref, wk_ref, wv_ref,
             xt_ref, q_ref, k_ref, v_ref):
        xt = jnp.dot(x_ref[...], wn_ref[...],
                     preferred_element_type=jnp.float32)
        xt_ref[...] = xt
        q_ref[...] = jnp.dot(xt, wq_ref[...],
                             preferred_element_type=jnp.float32)
        k_ref[...] = jnp.dot(xt, wk_ref[...],
                             preferred_element_type=jnp.float32)
        v_ref[...] = jnp.dot(xt, wv_ref[...],
                             preferred_element_type=jnp.float32)

    w_spec = pl.BlockSpec((DF, D), lambda i: (0, 0))
    r_spec = pl.BlockSpec((blk, D), lambda i: (i, 0))
    shp = jax.ShapeDtypeStruct((N, D), jnp.float32)
    return pl.pallas_call(
        body, grid=(N // blk,),
        in_specs=[pl.BlockSpec((blk, DF), lambda i: (i, 0)),
                  w_spec, w_spec, w_spec, w_spec],
        out_specs=[r_spec, r_spec, r_spec, r_spec],
        out_shape=[shp, shp, shp, shp],
    )(x, w_node, wq, wk, wv)


def _tc_edge_features(edge_attr, w_edge):
    blk = 4000

    def body(ea_ref, we_ref, ef_ref):
        ef_ref[...] = jnp.dot(ea_ref[...], we_ref[...],
                              preferred_element_type=jnp.float32)

    return pl.pallas_call(
        body, grid=(E // blk,),
        in_specs=[pl.BlockSpec((blk, DE), lambda i: (i, 0)),
                  pl.BlockSpec((DE, D), lambda i: (0, 0))],
        out_specs=pl.BlockSpec((blk, D), lambda i: (i, 0)),
        out_shape=jax.ShapeDtypeStruct((E, D), jnp.float32),
    )(edge_attr, w_edge)


def _tc_final(xt, agg_part, den2d):
    blk = 1000

    def body(xt_ref, ag_ref, dn_ref, o_ref):
        den128 = jnp.repeat(dn_ref[...], DH, axis=1)
        agg = (ag_ref[0] + ag_ref[1]) / (den128 + 1e-9)
        o_ref[...] = jnp.maximum(xt_ref[...] + agg, 0.0)

    return pl.pallas_call(
        body, grid=(N // blk,),
        in_specs=[pl.BlockSpec((blk, D), lambda i: (i, 0)),
                  pl.BlockSpec((2, blk, D), lambda i: (0, i, 0)),
                  pl.BlockSpec((blk, H), lambda i: (i, 0))],
        out_specs=pl.BlockSpec((blk, D), lambda i: (i, 0)),
        out_shape=jax.ShapeDtypeStruct((N, D), jnp.float32),
    )(xt, agg_part, den2d)


def _sc_logits(q, k, ef, src, dst):
    @functools.partial(
        pl.kernel,
        out_type=jax.ShapeDtypeStruct((H, E), jnp.float32),
        mesh=_mesh(),
        compiler_params=_sc_params(),
        scratch_types=[
            pltpu.VMEM((1, C), jnp.int32),
            pltpu.VMEM((1, C), jnp.int32),
            pltpu.VMEM((1, C), jnp.int32),
            pltpu.VMEM((1, C), jnp.int32),
            pltpu.VMEM((C, D), jnp.float32),
            pltpu.VMEM((C, D), jnp.float32),
            pltpu.VMEM((C, D), jnp.float32),
            pltpu.VMEM((C, D), jnp.float32),
            pltpu.VMEM((C, D), jnp.float32),
            pltpu.VMEM((C, D), jnp.float32),
            pltpu.VMEM((H, C), jnp.float32),
            pltpu.VMEM((H, C), jnp.float32),
            pltpu.SemaphoreType.DMA,
            pltpu.SemaphoreType.DMA,
            pltpu.SemaphoreType.DMA,
            pltpu.SemaphoreType.DMA,
            pltpu.SemaphoreType.DMA,
            pltpu.SemaphoreType.DMA,
        ],
    )
    def kern(q_hbm, k_hbm, ef_hbm, src_hbm, dst_hbm, lo_hbm,
             di0, di1, si0, si1, qb0, qb1, kb0, kb1, eb0, eb1, lb0, lb1,
             gsem0, gsem1, isem0, isem1, lsem0, lsem1):
        wid = _worker_id()
        nmy = jnp.where(wid < NCHUNKS - (NCH_HI - 1) * NW,
                        NCH_HI, NCH_HI - 1)
        lane = lax.iota(jnp.int32, L)
        lastm = lane == (L - 1)
        slots = ((di0, si0, qb0, kb0, eb0, lb0, gsem0, isem0, lsem0),
                 (di1, si1, qb1, kb1, eb1, lb1, gsem1, isem1, lsem1))

        def issue_idx(s, t):
            di, si, _, _, _, _, _, isem, _ = slots[s]
            off = (wid + NW * t) * C
            pltpu.async_copy(dst_hbm.at[pl.ds(off, C)], di.at[0], isem)
            pltpu.async_copy(src_hbm.at[pl.ds(off, C)], si.at[0], isem)

        def issue_gather(s, t):
            di, si, qb, kb, eb, _, gsem, isem, _ = slots[s]
            off = (wid + NW * t) * C
            pltpu.make_async_copy(dst_hbm.at[pl.ds(off, C)], di.at[0],
                                  isem).wait()
            pltpu.make_async_copy(src_hbm.at[pl.ds(off, C)], si.at[0],
                                  isem).wait()
            pltpu.async_copy(q_hbm.at[di.at[0]], qb, gsem)
            pltpu.async_copy(k_hbm.at[si.at[0]], kb, gsem)
            pltpu.async_copy(ef_hbm.at[pl.ds(off, C)], eb, gsem)

        def wait_gathers(s, t):
            di, si, qb, kb, eb, _, gsem, _, _ = slots[s]
            off = (wid + NW * t) * C
            pltpu.make_async_copy(q_hbm.at[di.at[0]], qb, gsem).wait()
            pltpu.make_async_copy(k_hbm.at[si.at[0]], kb, gsem).wait()
            pltpu.make_async_copy(ef_hbm.at[pl.ds(off, C)], eb,
                                  gsem).wait()

        def compute(s, t):
            _, _, qb, kb, eb, lb, _, _, lsem = slots[s]
            off = (wid + NW * t) * C

            @pl.when(t >= 2)
            def _():
                pltpu.make_async_copy(lb, lo_hbm.at[:, pl.ds(off, C)],
                                      lsem).wait()

            @pl.loop(0, C)
            def _(e):
                he = lane * 0 + e
                for h in range(H):
                    j0, j1 = 2 * h, 2 * h + 1
                    t_ = (qb[e, pl.ds(L * j0, L)]
                          * (kb[e, pl.ds(L * j0, L)]
                             + eb[e, pl.ds(L * j0, L)])
                          + qb[e, pl.ds(L * j1, L)]
                          * (kb[e, pl.ds(L * j1, L)]
                             + eb[e, pl.ds(L * j1, L)]))
                    cs = plsc.cumsum(t_) * INV_SQRT_DH
                    plsc.store_scatter(lb, [lane * 0 + h, he], cs,
                                       mask=lastm)

            pltpu.async_copy(lb, lo_hbm.at[:, pl.ds(off, C)], lsem)

        @pl.when(0 < nmy)
        def _():
            issue_idx(0, 0)

        @pl.when(1 < nmy)
        def _():
            issue_idx(1, 1)

        @pl.when(0 < nmy)
        def _():
            issue_gather(0, 0)

        @pl.loop(0, (NCH_HI + 1) // 2)
        def _(i):
            t0 = 2 * i
            t1 = 2 * i + 1

            @pl.when(t1 < nmy)
            def _():
                issue_gather(1, t1)

            @pl.when(t0 < nmy)
            def _():
                wait_gathers(0, t0)

            @pl.when(t0 + 2 < nmy)
            def _():
                issue_idx(0, t0 + 2)

            @pl.when(t0 < nmy)
            def _():
                compute(0, t0)

            @pl.when(t1 + 1 < nmy)
            def _():
                issue_gather(0, t1 + 1)

            @pl.when(t1 < nmy)
            def _():
                wait_gathers(1, t1)

            @pl.when(t1 + 2 < nmy)
            def _():
                issue_idx(1, t1 + 2)

            @pl.when(t1 < nmy)
            def _():
                compute(1, t1)

        @pl.when(nmy >= 2)
        def _():
            pltpu.make_async_copy(lb0, lo_hbm.at[:, pl.ds(0, C)],
                                  lsem0).wait()
            pltpu.make_async_copy(lb1, lo_hbm.at[:, pl.ds(0, C)],
                                  lsem1).wait()

    return kern(q, k, ef, src, dst)


def _sc_segmax(lo, dst):
    @functools.partial(
        pl.kernel,
        out_type=jax.ShapeDtypeStruct((NW * NPH,), jnp.float32),
        mesh=_mesh(),
        compiler_params=_sc_params(),
        scratch_types=[
            pltpu.VMEM((NPH,), jnp.float32),
            pltpu.VMEM((H, C), jnp.float32),
            pltpu.VMEM((1, C), jnp.int32),
        ],
    )
    def kern(lo_hbm, dst_hbm, mp_hbm, m_loc, lb, di):
        wid = _worker_id()

        @pl.loop(0, NPH, step=L)
        def _(i):
            m_loc[pl.ds(i, L)] = jnp.full((L,), NEG_INF, jnp.float32)

        def chunk(off):
            pltpu.sync_copy(dst_hbm.at[pl.ds(off, C)], di.at[0])
            pltpu.sync_copy(lo_hbm.at[:, pl.ds(off, C)], lb)
            for g in range(C // L):
                dstv = di[0, pl.ds(L * g, L)]
                vals = [lb[h, pl.ds(L * g, L)] for h in range(H)]
                sdst, firstm, svals = _dedup_combine(dstv, vals, True)
                b4 = sdst * H
                for h in range(H):
                    idx = b4 + h
                    cur = plsc.load_gather(m_loc, [idx])
                    plsc.store_scatter(m_loc, [idx],
                                       jnp.maximum(cur, svals[h]),
                                       mask=firstm)

        _foreach_chunk(wid, chunk)
        pltpu.sync_copy(m_loc, mp_hbm.at[pl.ds(wid * NPH, NPH)])

    return kern(lo, dst)


def _sc_combine(parts, is_max):
    cw = NPH // NW

    @functools.partial(
        pl.kernel,
        out_type=jax.ShapeDtypeStruct((NPH,), jnp.float32),
        mesh=_mesh(),
        compiler_params=_sc_params(),
        scratch_types=[
            pltpu.VMEM((cw,), jnp.float32),
            pltpu.VMEM((cw,), jnp.float32),
        ],
    )
    def kern(parts_hbm, out_hbm, acc, tmp):
        wid = _worker_id()
        off = wid * cw
        comb = jnp.maximum if is_max else jnp.add
        pltpu.sync_copy(parts_hbm.at[pl.ds(off, cw)], acc)
        for t in range(1, NW):
            pltpu.sync_copy(parts_hbm.at[pl.ds(t * NPH + off, cw)], tmp)

            @pl.loop(0, cw, step=L)
            def _(i):
                acc[pl.ds(i, L)] = comb(acc[pl.ds(i, L)], tmp[pl.ds(i, L)])

        pltpu.sync_copy(acc, out_hbm.at[pl.ds(off, cw)])

    return kern(parts)


def _sc_exp_den(lo, dst, m_final):
    @functools.partial(
        pl.kernel,
        out_type=(jax.ShapeDtypeStruct((E, H), jnp.float32),
                  jax.ShapeDtypeStruct((NW * NPH,), jnp.float32)),
        mesh=_mesh(),
        compiler_params=_sc_params(),
        scratch_types=[
            pltpu.VMEM((NPH,), jnp.float32),
            pltpu.VMEM((NPH,), jnp.float32),
            pltpu.VMEM((H, C), jnp.float32),
            pltpu.VMEM((C, H), jnp.float32),
            pltpu.VMEM((1, C), jnp.int32),
        ],
    )
    def kern(lo_hbm, dst_hbm, m_hbm, ex_hbm, dp_hbm,
             m_loc, den_loc, lb, exb, di):
        lane = lax.iota(jnp.int32, L)
        wid = _worker_id()
        pltpu.sync_copy(m_hbm, m_loc)

        @pl.loop(0, NPH, step=L)
        def _(i):
            den_loc[pl.ds(i, L)] = jnp.zeros((L,), jnp.float32)

        def chunk(off):
            pltpu.sync_copy(dst_hbm.at[pl.ds(off, C)], di.at[0])
            pltpu.sync_copy(lo_hbm.at[:, pl.ds(off, C)], lb)
            for g in range(C // L):
                dstv = di[0, pl.ds(L * g, L)]
                b4 = dstv * H
                exs = []
                erow = lane + L * g
                for h in range(H):
                    mg = plsc.load_gather(m_loc, [b4 + h])
                    exv = jnp.exp(lb[h, pl.ds(L * g, L)] - mg)
                    plsc.store_scatter(exb, [erow, lane * 0 + h], exv)
                    exs.append(exv)
                sdst, firstm, svals = _dedup_combine(dstv, exs, False)
                sb4 = sdst * H
                for h in range(H):
                    idx = sb4 + h
                    cur = plsc.load_gather(den_loc, [idx])
                    plsc.store_scatter(den_loc, [idx], cur + svals[h],
                                       mask=firstm)
            pltpu.sync_copy(exb, ex_hbm.at[pl.ds(off, C)])

        _foreach_chunk(wid, chunk)
        pltpu.sync_copy(den_loc, dp_hbm.at[pl.ds(wid * NPH, NPH)])

    return kern(lo, dst, m_final)


def _sc_aggregate(v, ef, ex, src, dst):
    C4 = 64
    nch4 = E // C4
    hi4 = -(-nch4 // NW)

    @functools.partial(
        pl.kernel,
        out_type=jax.ShapeDtypeStruct((2, N, D), jnp.float32),
        mesh=_mesh(),
        compiler_params=_sc_params(),
        scratch_types=[
            pltpu.VMEM((1, C4), jnp.int32),
            pltpu.VMEM((1, C4), jnp.int32),
            pltpu.VMEM((1, C4), jnp.int32),
            pltpu.VMEM((1, C4), jnp.int32),
            pltpu.VMEM((C4, D), jnp.float32),
            pltpu.VMEM((C4, D), jnp.float32),
            pltpu.VMEM((C4, D), jnp.float32),
            pltpu.VMEM((C4, D), jnp.float32),
            pltpu.VMEM((C4, H), jnp.float32),
            pltpu.VMEM((C4, H), jnp.float32),
            pltpu.VMEM((1, C4), jnp.int32),
            pltpu.SemaphoreType.DMA,
            pltpu.SemaphoreType.DMA,
            pltpu.SemaphoreType.DMA,
            pltpu.SemaphoreType.DMA,
            pltpu.VMEM_SHARED((N, D), jnp.float32),
        ],
    )
    def kern(v_hbm, ef_hbm, ex_hbm, src_hbm, dst_hbm, ag_hbm,
             di0, di1, si0, si1, vb0, vb1, eb0, eb1, xb0, xb1, dsc,
             sem0, sem1, isem0, isem1, agg_sp):
        cid = lax.axis_index("c")
        sid = lax.axis_index("s")
        wid = _worker_id()
        lane = lax.iota(jnp.int32, L)
        nmy = jnp.where(wid < nch4 - (hi4 - 1) * NW, hi4, hi4 - 1)
        slots = ((di0, si0, vb0, eb0, xb0, sem0, isem0),
                 (di1, si1, vb1, eb1, xb1, sem1, isem1))

        @pl.loop(0, C4)
        def _(r):
            @pl.loop(0, D, step=L)
            def _(c0):
                vb0[r, pl.ds(c0, L)] = jnp.zeros((L,), jnp.float32)

        RT = 624
        row0 = sid * RT

        @pl.when(sid < 15)
        def _():
            for b in range(RT // C4):
                pltpu.sync_copy(vb0, agg_sp.at[pl.ds(row0 + b * C4, C4)])
            pltpu.sync_copy(vb0.at[pl.ds(0, RT - (RT // C4) * C4)],
                            agg_sp.at[pl.ds(row0 + (RT // C4) * C4,
                                            RT - (RT // C4) * C4)])

        @pl.when(sid == 15)
        def _():
            for b in range((N - 15 * RT) // C4):
                pltpu.sync_copy(vb0, agg_sp.at[pl.ds(row0 + b * C4, C4)])

        plsc.subcore_barrier()

        def issue_idx(s, t):
            di, si, _, _, _, _, isem = slots[s]
            off = (wid + NW * t) * C4
            pltpu.async_copy(dst_hbm.at[pl.ds(off, C4)], di.at[0], isem)
            pltpu.async_copy(src_hbm.at[pl.ds(off, C4)], si.at[0], isem)

        def issue_fetch(s, t):
            di, si, vb, eb, xb, gsem, isem = slots[s]
            off = (wid + NW * t) * C4
            pltpu.make_async_copy(dst_hbm.at[pl.ds(off, C4)], di.at[0],
                                  isem).wait()
            pltpu.make_async_copy(src_hbm.at[pl.ds(off, C4)], si.at[0],
                                  isem).wait()
            pltpu.async_copy(v_hbm.at[si.at[0]], vb, gsem)
            pltpu.async_copy(ef_hbm.at[pl.ds(off, C4)], eb, gsem)
            pltpu.async_copy(ex_hbm.at[pl.ds(off, C4)], xb, gsem)

        def wait_fetch(s, t):
            di, si, vb, eb, xb, gsem, isem = slots[s]
            off = (wid + NW * t) * C4
            pltpu.make_async_copy(v_hbm.at[si.at[0]], vb, gsem).wait()
            pltpu.make_async_copy(ef_hbm.at[pl.ds(off, C4)], eb,
                                  gsem).wait()
            pltpu.make_async_copy(ex_hbm.at[pl.ds(off, C4)], xb,
                                  gsem).wait()
            for j in range(C4 // L):
                dsc[0, pl.ds(L * j, L)] = di[0, pl.ds(L * j, L)]

        def compute_scatter(s, t):
            di, si, vb, eb, xb, gsem, isem = slots[s]
            for g in range(C4 // L):
                erow = lane + L * g
                exs = [plsc.load_gather(xb, [erow, lane * 0 + h])
                       for h in range(H)]

                @pl.loop(0, L)
                def _(e16):
                    row = L * g + e16
                    sel = lane * 0 + e16
                    bc = [_vshuffle(exs[h], sel) for h in range(H)]
                    for j in range(D // L):
                        vb[row, pl.ds(L * j, L)] = (
                            (vb[row, pl.ds(L * j, L)]
                             + eb[row, pl.ds(L * j, L)]) * bc[j // 2])

            pltpu.sync_copy(vb, agg_sp.at[dsc.at[0]], add=True)

        @pl.when(0 < nmy)
        def _():
            issue_idx(0, 0)

        @pl.when(1 < nmy)
        def _():
            issue_idx(1, 1)

        @pl.when(0 < nmy)
        def _():
            issue_fetch(0, 0)

        @pl.loop(0, (hi4 + 1) // 2)
        def _(i):
            t0 = 2 * i
            t1 = 2 * i + 1

            @pl.when(t1 < nmy)
            def _():
                issue_fetch(1, t1)

            @pl.when(t0 < nmy)
            def _():
                wait_fetch(0, t0)

            @pl.when(t0 + 2 < nmy)
            def _():
                issue_idx(0, t0 + 2)

            @pl.when(t0 < nmy)
            def _():
                compute_scatter(0, t0)

            @pl.when(t1 + 1 < nmy)
            def _():
                issue_fetch(0, t1 + 1)

            @pl.when(t1 < nmy)
            def _():
                wait_fetch(1, t1)

            @pl.when(t1 + 2 < nmy)
            def _():
                issue_idx(1, t1 + 2)

            @pl.when(t1 < nmy)
            def _():
                compute_scatter(1, t1)

        plsc.subcore_barrier()

        @pl.when(sid < 15)
        def _():
            for b in range(RT // C):
                pltpu.sync_copy(agg_sp.at[pl.ds(row0 + b * C, C)],
                                ag_hbm.at[cid, pl.ds(row0 + b * C, C)])
            rem = RT - (RT // C) * C
            pltpu.sync_copy(
                agg_sp.at[pl.ds(row0 + (RT // C) * C, rem)],
                ag_hbm.at[cid, pl.ds(row0 + (RT // C) * C, rem)])

        @pl.when(sid == 15)
        def _():
            nv = N - 15 * RT
            for b in range(nv // C):
                pltpu.sync_copy(agg_sp.at[pl.ds(row0 + b * C, C)],
                                ag_hbm.at[cid, pl.ds(row0 + b * C, C)])

    return kern(v, ef, ex, src, dst)


def kernel(x, edge_index, edge_attr, W_node, W_edge, Wq, Wk, Wv):
    src = edge_index[0]
    dst = edge_index[1]
    xt, q, k, v = _tc_tables(x, W_node, Wq, Wk, Wv)
    ef = _tc_edge_features(edge_attr, W_edge)
    lo = _sc_logits(q, k, ef, src, dst)
    m_parts = _sc_segmax(lo, dst)
    m_final = _sc_combine(m_parts, True)
    ex, den_parts = _sc_exp_den(lo, dst, m_final)
    den_final = _sc_combine(den_parts, False)
    agg_part = _sc_aggregate(v, ef, ex, src, dst)
    den2d = den_final[:N * H].reshape(N, H)
    return _tc_final(xt, agg_part, den2d)


# speedup vs baseline: 3.9465x; 1.1137x over previous
import dataclasses
import functools

import jax
import jax.numpy as jnp
from jax import lax
from jax.experimental import pallas as pl
from jax.experimental.pallas import tpu as pltpu
from jax.experimental.pallas import tpu_sc as plsc

N = 10000
E = 320000
DF = 128
DE = 16
D = 128
H = 4
DH = D // H
L = 16
NW = 32
C = 128
NCHUNKS = E // C
NCH_HI = -(-NCHUNKS // NW)
NP = 10240
NPH = NP * H
SLICE = NPH // 16
AGG_ROWS = NP // 16
INV_SQRT_DH = 1.0 / (DH ** 0.5)
NEG_INF = float("-inf")


def _mesh():
    return plsc.VectorSubcoreMesh(core_axis_name="c", subcore_axis_name="s")


def _sc_params():
    cp = pltpu.CompilerParams()
    if "needs_layout_passes" in pltpu.CompilerParams.__dataclass_fields__:
        cp = dataclasses.replace(cp, needs_layout_passes=False)
    return cp


def _worker_id():
    return lax.axis_index("s") * 2 + lax.axis_index("c")


def _vshuffle(x, idx):
    dn = lax.GatherDimensionNumbers(
        offset_dims=(), collapsed_slice_dims=(0,), start_index_map=(0,))
    return lax.gather(x, idx[:, None], dn, (1,),
                      mode=lax.GatherScatterMode.PROMISE_IN_BOUNDS)


def _dedup_combine(dstv, vals, is_max):
    lane = lax.iota(jnp.int32, L)
    sdst, perm = plsc.sort_key_val(dstv, lane)
    steps = []
    for d in (1, 2, 4, 8):
        nb = jnp.minimum(lane + d, L - 1)
        same = (_vshuffle(sdst, nb) == sdst) & (lane < L - d)
        steps.append((nb, same))
    prev = jnp.maximum(lane - 1, 0)
    firstm = (lane == 0) | (_vshuffle(sdst, prev) != sdst)
    ident = NEG_INF if is_max else 0.0
    comb = jnp.maximum if is_max else jnp.add
    out = []
    for v in vals:
        sv = _vshuffle(v, perm)
        for nb, same in steps:
            sv = comb(sv, jnp.where(same, _vshuffle(sv, nb), ident))
        out.append(sv)
    return sdst, firstm, out


def _foreach_chunk(wid, fn):
    @pl.loop(0, NCH_HI)
    def _(t):
        cidx = wid + NW * t

        @pl.when(cidx < NCHUNKS)
        def _():
            fn(cidx * C)


def _tc_tables(x, w_node, wq, wk, wv):
    blk = 1000

    def body(x_ref, wn_ref, wq_ref, wk_ref, wv_ref,
             xt_ref, q_ref, k_ref, v_ref):
        xt = jnp.dot(x_ref[...], wn_ref[...],
                     preferred_element_type=jnp.float32)
        xt_ref[...] = xt
        q_ref[...] = jnp.dot(xt, wq_ref[...],
                             preferred_element_type=jnp.float32)
        k_ref[...] = jnp.dot(xt, wk_ref[...],
                             preferred_element_type=jnp.float32)
        v_ref[...] = jnp.dot(xt, wv_ref[...],
                             preferred_element_type=jnp.float32)

    w_spec = pl.BlockSpec((DF, D), lambda i: (0, 0))
    r_spec = pl.BlockSpec((blk, D), lambda i: (i, 0))
    shp = jax.ShapeDtypeStruct((N, D), jnp.float32)
    return pl.pallas_call(
        body, grid=(N // blk,),
        in_specs=[pl.BlockSpec((blk, DF), lambda i: (i, 0)),
                  w_spec, w_spec, w_spec, w_spec],
        out_specs=[r_spec, r_spec, r_spec, r_spec],
        out_shape=[shp, shp, shp, shp],
    )(x, w_node, wq, wk, wv)


def _tc_edge_features(edge_attr, w_edge):
    blk = 4000

    def body(ea_ref, we_ref, ef_ref):
        ef_ref[...] = jnp.dot(ea_ref[...], we_ref[...],
                              preferred_element_type=jnp.float32)

    return pl.pallas_call(
        body, grid=(E // blk,),
        in_specs=[pl.BlockSpec((blk, DE), lambda i: (i, 0)),
                  pl.BlockSpec((DE, D), lambda i: (0, 0))],
        out_specs=pl.BlockSpec((blk, D), lambda i: (i, 0)),
        out_shape=jax.ShapeDtypeStruct((E, D), jnp.float32),
    )(edge_attr, w_edge)


def _tc_final(xt, agg_part, den2d):
    blk = 1000

    def body(xt_ref, ag_ref, dn_ref, o_ref):
        den128 = jnp.repeat(dn_ref[...], DH, axis=1)
        agg = (ag_ref[0] + ag_ref[1]) / (den128 + 1e-9)
        o_ref[...] = jnp.maximum(xt_ref[...] + agg, 0.0)

    return pl.pallas_call(
        body, grid=(N // blk,),
        in_specs=[pl.BlockSpec((blk, D), lambda i: (i, 0)),
                  pl.BlockSpec((2, blk, D), lambda i: (0, i, 0)),
                  pl.BlockSpec((blk, H), lambda i: (i, 0))],
        out_specs=pl.BlockSpec((blk, D), lambda i: (i, 0)),
        out_shape=jax.ShapeDtypeStruct((N, D), jnp.float32),
    )(xt, agg_part, den2d)


def _sc_logits(q, k, ef, src, dst):
    @functools.partial(
        pl.kernel,
        out_type=jax.ShapeDtypeStruct((H, E), jnp.float32),
        mesh=_mesh(),
        compiler_params=_sc_params(),
        scratch_types=[
            pltpu.VMEM((1, C), jnp.int32),
            pltpu.VMEM((1, C), jnp.int32),
            pltpu.VMEM((1, C), jnp.int32),
            pltpu.VMEM((1, C), jnp.int32),
            pltpu.VMEM((C, D), jnp.float32),
            pltpu.VMEM((C, D), jnp.float32),
            pltpu.VMEM((C, D), jnp.float32),
            pltpu.VMEM((C, D), jnp.float32),
            pltpu.VMEM((C, D), jnp.float32),
            pltpu.VMEM((C, D), jnp.float32),
            pltpu.VMEM((H, C), jnp.float32),
            pltpu.VMEM((H, C), jnp.float32),
            pltpu.SemaphoreType.DMA,
            pltpu.SemaphoreType.DMA,
            pltpu.SemaphoreType.DMA,
            pltpu.SemaphoreType.DMA,
            pltpu.SemaphoreType.DMA,
            pltpu.SemaphoreType.DMA,
        ],
    )
    def kern(q_hbm, k_hbm, ef_hbm, src_hbm, dst_hbm, lo_hbm,
             di0, di1, si0, si1, qb0, qb1, kb0, kb1, eb0, eb1, lb0, lb1,
             gsem0, gsem1, isem0, isem1, lsem0, lsem1):
        wid = _worker_id()
        nmy = jnp.where(wid < NCHUNKS - (NCH_HI - 1) * NW,
                        NCH_HI, NCH_HI - 1)
        lane = lax.iota(jnp.int32, L)
        lastm = lane == (L - 1)
        slots = ((di0, si0, qb0, kb0, eb0, lb0, gsem0, isem0, lsem0),
                 (di1, si1, qb1, kb1, eb1, lb1, gsem1, isem1, lsem1))

        def issue_idx(s, t):
            di, si, _, _, _, _, _, isem, _ = slots[s]
            off = (wid + NW * t) * C
            pltpu.async_copy(dst_hbm.at[pl.ds(off, C)], di.at[0], isem)
            pltpu.async_copy(src_hbm.at[pl.ds(off, C)], si.at[0], isem)

        def issue_gather(s, t):
            di, si, qb, kb, eb, _, gsem, isem, _ = slots[s]
            off = (wid + NW * t) * C
            pltpu.make_async_copy(dst_hbm.at[pl.ds(off, C)], di.at[0],
                                  isem).wait()
            pltpu.make_async_copy(src_hbm.at[pl.ds(off, C)], si.at[0],
                                  isem).wait()
            pltpu.async_copy(q_hbm.at[di.at[0]], qb, gsem)
            pltpu.async_copy(k_hbm.at[si.at[0]], kb, gsem)
            pltpu.async_copy(ef_hbm.at[pl.ds(off, C)], eb, gsem)

        def wait_gathers(s, t):
            di, si, qb, kb, eb, _, gsem, _, _ = slots[s]
            off = (wid + NW * t) * C
            pltpu.make_async_copy(q_hbm.at[di.at[0]], qb, gsem).wait()
            pltpu.make_async_copy(k_hbm.at[si.at[0]], kb, gsem).wait()
            pltpu.make_async_copy(ef_hbm.at[pl.ds(off, C)], eb,
                                  gsem).wait()

        def compute(s, t):
            _, _, qb, kb, eb, lb, _, _, lsem = slots[s]
            off = (wid + NW * t) * C

            @pl.when(t >= 2)
            def _():
                pltpu.make_async_copy(lb, lo_hbm.at[:, pl.ds(off, C)],
                                      lsem).wait()

            @pl.loop(0, C)
            def _(e):
                he = lane * 0 + e
                for h in range(H):
                    j0, j1 = 2 * h, 2 * h + 1
                    t_ = (qb[e, pl.ds(L * j0, L)]
                          * (kb[e, pl.ds(L * j0, L)]
                             + eb[e, pl.ds(L * j0, L)])
                          + qb[e, pl.ds(L * j1, L)]
                          * (kb[e, pl.ds(L * j1, L)]
                             + eb[e, pl.ds(L * j1, L)]))
                    cs = plsc.cumsum(t_) * INV_SQRT_DH
                    plsc.store_scatter(lb, [lane * 0 + h, he], cs,
                                       mask=lastm)

            pltpu.async_copy(lb, lo_hbm.at[:, pl.ds(off, C)], lsem)

        @pl.when(0 < nmy)
        def _():
            issue_idx(0, 0)

        @pl.when(1 < nmy)
        def _():
            issue_idx(1, 1)

        @pl.when(0 < nmy)
        def _():
            issue_gather(0, 0)

        @pl.loop(0, (NCH_HI + 1) // 2)
        def _(i):
            t0 = 2 * i
            t1 = 2 * i + 1

            @pl.when(t1 < nmy)
            def _():
                issue_gather(1, t1)

            @pl.when(t0 < nmy)
            def _():
                wait_gathers(0, t0)

            @pl.when(t0 + 2 < nmy)
            def _():
                issue_idx(0, t0 + 2)

            @pl.when(t0 < nmy)
            def _():
                compute(0, t0)

            @pl.when(t1 + 1 < nmy)
            def _():
                issue_gather(0, t1 + 1)

            @pl.when(t1 < nmy)
            def _():
                wait_gathers(1, t1)

            @pl.when(t1 + 2 < nmy)
            def _():
                issue_idx(1, t1 + 2)

            @pl.when(t1 < nmy)
            def _():
                compute(1, t1)

        @pl.when(nmy >= 2)
        def _():
            pltpu.make_async_copy(lb0, lo_hbm.at[:, pl.ds(0, C)],
                                  lsem0).wait()
            pltpu.make_async_copy(lb1, lo_hbm.at[:, pl.ds(0, C)],
                                  lsem1).wait()

    return kern(q, k, ef, src, dst)


def _sc_segmax(lo, dst):
    @functools.partial(
        pl.kernel,
        out_type=jax.ShapeDtypeStruct((NW * NPH,), jnp.float32),
        mesh=_mesh(),
        compiler_params=_sc_params(),
        scratch_types=[
            pltpu.VMEM((NPH,), jnp.float32),
            pltpu.VMEM((H, C), jnp.float32),
            pltpu.VMEM((H, C), jnp.float32),
            pltpu.VMEM((1, C), jnp.int32),
            pltpu.VMEM((1, C), jnp.int32),
            pltpu.SemaphoreType.DMA,
            pltpu.SemaphoreType.DMA,
        ],
    )
    def kern(lo_hbm, dst_hbm, mp_hbm, m_loc, lb0, lb1, di0, di1,
             gsem0, gsem1):
        wid = _worker_id()
        nmy = jnp.where(wid < NCHUNKS - (NCH_HI - 1) * NW,
                        NCH_HI, NCH_HI - 1)
        slots = ((di0, lb0, gsem0), (di1, lb1, gsem1))

        @pl.loop(0, NPH, step=L)
        def _(i):
            m_loc[pl.ds(i, L)] = jnp.full((L,), NEG_INF, jnp.float32)

        def issue(s, t):
            di, lb, gsem = slots[s]
            off = (wid + NW * t) * C
            pltpu.async_copy(dst_hbm.at[pl.ds(off, C)], di.at[0], gsem)
            pltpu.async_copy(lo_hbm.at[:, pl.ds(off, C)], lb, gsem)

        def consume(s, t):
            di, lb, gsem = slots[s]
            off = (wid + NW * t) * C
            pltpu.make_async_copy(dst_hbm.at[pl.ds(off, C)], di.at[0],
                                  gsem).wait()
            pltpu.make_async_copy(lo_hbm.at[:, pl.ds(off, C)], lb,
                                  gsem).wait()
            for g in range(C // L):
                dstv = di[0, pl.ds(L * g, L)]
                vals = [lb[h, pl.ds(L * g, L)] for h in range(H)]
                sdst, firstm, svals = _dedup_combine(dstv, vals, True)
                b4 = sdst * H
                for h in range(H):
                    idx = b4 + h
                    cur = plsc.load_gather(m_loc, [idx])
                    plsc.store_scatter(m_loc, [idx],
                                       jnp.maximum(cur, svals[h]),
                                       mask=firstm)

        @pl.when(0 < nmy)
        def _():
            issue(0, 0)

        @pl.loop(0, (NCH_HI + 1) // 2)
        def _(i):
            t0 = 2 * i
            t1 = 2 * i + 1

            @pl.when(t1 < nmy)
            def _():
                issue(1, t1)

            @pl.when(t0 < nmy)
            def _():
                consume(0, t0)

            @pl.when(t1 + 1 < nmy)
            def _():
                issue(0, t1 + 1)

            @pl.when(t1 < nmy)
            def _():
                consume(1, t1)

        pltpu.sync_copy(m_loc, mp_hbm.at[pl.ds(wid * NPH, NPH)])

    return kern(lo, dst)


def _sc_combine(parts, is_max):
    cw = NPH // NW

    @functools.partial(
        pl.kernel,
        out_type=jax.ShapeDtypeStruct((NPH,), jnp.float32),
        mesh=_mesh(),
        compiler_params=_sc_params(),
        scratch_types=[
            pltpu.VMEM((cw,), jnp.float32),
            pltpu.VMEM((cw,), jnp.float32),
        ],
    )
    def kern(parts_hbm, out_hbm, acc, tmp):
        wid = _worker_id()
        off = wid * cw
        comb = jnp.maximum if is_max else jnp.add
        pltpu.sync_copy(parts_hbm.at[pl.ds(off, cw)], acc)
        for t in range(1, NW):
            pltpu.sync_copy(parts_hbm.at[pl.ds(t * NPH + off, cw)], tmp)

            @pl.loop(0, cw, step=L)
            def _(i):
                acc[pl.ds(i, L)] = comb(acc[pl.ds(i, L)], tmp[pl.ds(i, L)])

        pltpu.sync_copy(acc, out_hbm.at[pl.ds(off, cw)])

    return kern(parts)


def _sc_exp_den(lo, dst, m_final):
    @functools.partial(
        pl.kernel,
        out_type=(jax.ShapeDtypeStruct((E, H), jnp.float32),
                  jax.ShapeDtypeStruct((NW * NPH,), jnp.float32)),
        mesh=_mesh(),
        compiler_params=_sc_params(),
        scratch_types=[
            pltpu.VMEM((NPH,), jnp.float32),
            pltpu.VMEM((NPH,), jnp.float32),
            pltpu.VMEM((H, C), jnp.float32),
            pltpu.VMEM((H, C), jnp.float32),
            pltpu.VMEM((C, H), jnp.float32),
            pltpu.VMEM((C, H), jnp.float32),
            pltpu.VMEM((1, C), jnp.int32),
            pltpu.VMEM((1, C), jnp.int32),
            pltpu.SemaphoreType.DMA,
            pltpu.SemaphoreType.DMA,
            pltpu.SemaphoreType.DMA,
            pltpu.SemaphoreType.DMA,
        ],
    )
    def kern(lo_hbm, dst_hbm, m_hbm, ex_hbm, dp_hbm,
             m_loc, den_loc, lb0, lb1, exb0, exb1, di0, di1,
             gsem0, gsem1, xsem0, xsem1):
        wid = _worker_id()
        lane = lax.iota(jnp.int32, L)
        nmy = jnp.where(wid < NCHUNKS - (NCH_HI - 1) * NW,
                        NCH_HI, NCH_HI - 1)
        slots = ((di0, lb0, exb0, gsem0, xsem0),
                 (di1, lb1, exb1, gsem1, xsem1))

        pltpu.sync_copy(m_hbm, m_loc)

        @pl.loop(0, NPH, step=L)
        def _(i):
            den_loc[pl.ds(i, L)] = jnp.zeros((L,), jnp.float32)

        def issue(s, t):
            di, lb, exb, gsem, xsem = slots[s]
            off = (wid + NW * t) * C
            pltpu.async_copy(dst_hbm.at[pl.ds(off, C)], di.at[0], gsem)
            pltpu.async_copy(lo_hbm.at[:, pl.ds(off, C)], lb, gsem)

        def consume(s, t):
            di, lb, exb, gsem, xsem = slots[s]
            off = (wid + NW * t) * C
            pltpu.make_async_copy(dst_hbm.at[pl.ds(off, C)], di.at[0],
                                  gsem).wait()
            pltpu.make_async_copy(lo_hbm.at[:, pl.ds(off, C)], lb,
                                  gsem).wait()

            @pl.when(t >= 2)
            def _():
                pltpu.make_async_copy(exb, ex_hbm.at[pl.ds(off, C)],
                                      xsem).wait()

            for g in range(C // L):
                dstv = di[0, pl.ds(L * g, L)]
                b4 = dstv * H
                erow = lane + L * g
                exs = []
                for h in range(H):
                    mg = plsc.load_gather(m_loc, [b4 + h])
                    exv = jnp.exp(lb[h, pl.ds(L * g, L)] - mg)
                    plsc.store_scatter(exb, [erow, lane * 0 + h], exv)
                    exs.append(exv)
                sdst, firstm, svals = _dedup_combine(dstv, exs, False)
                sb4 = sdst * H
                for h in range(H):
                    idx = sb4 + h
                    cur = plsc.load_gather(den_loc, [idx])
                    plsc.store_scatter(den_loc, [idx], cur + svals[h],
                                       mask=firstm)

            pltpu.async_copy(exb, ex_hbm.at[pl.ds(off, C)], xsem)

        @pl.when(0 < nmy)
        def _():
            issue(0, 0)

        @pl.loop(0, (NCH_HI + 1) // 2)
        def _(i):
            t0 = 2 * i
            t1 = 2 * i + 1

            @pl.when(t1 < nmy)
            def _():
                issue(1, t1)

            @pl.when(t0 < nmy)
            def _():
                consume(0, t0)

            @pl.when(t1 + 1 < nmy)
            def _():
                issue(0, t1 + 1)

            @pl.when(t1 < nmy)
            def _():
                consume(1, t1)

        @pl.when(nmy >= 2)
        def _():
            pltpu.make_async_copy(exb0, ex_hbm.at[pl.ds(0, C)],
                                  xsem0).wait()
            pltpu.make_async_copy(exb1, ex_hbm.at[pl.ds(0, C)],
                                  xsem1).wait()

        pltpu.sync_copy(den_loc, dp_hbm.at[pl.ds(wid * NPH, NPH)])

    return kern(lo, dst, m_final)


def _sc_aggregate(v, ef, ex, src, dst):
    C4 = 64
    nch4 = E // C4
    hi4 = -(-nch4 // NW)

    @functools.partial(
        pl.kernel,
        out_type=jax.ShapeDtypeStruct((2, N, D), jnp.float32),
        mesh=_mesh(),
        compiler_params=_sc_params(),
        scratch_types=[
            pltpu.VMEM((1, C4), jnp.int32),
            pltpu.VMEM((1, C4), jnp.int32),
            pltpu.VMEM((1, C4), jnp.int32),
            pltpu.VMEM((1, C4), jnp.int32),
            pltpu.VMEM((C4, D), jnp.float32),
            pltpu.VMEM((C4, D), jnp.float32),
            pltpu.VMEM((C4, D), jnp.float32),
            pltpu.VMEM((C4, D), jnp.float32),
            pltpu.VMEM((C4, H), jnp.float32),
            pltpu.VMEM((C4, H), jnp.float32),
            pltpu.VMEM((1, C4), jnp.int32),
            pltpu.SemaphoreType.DMA,
            pltpu.SemaphoreType.DMA,
            pltpu.SemaphoreType.DMA,
            pltpu.SemaphoreType.DMA,
            pltpu.VMEM_SHARED((N, D), jnp.float32),
        ],
    )
    def kern(v_hbm, ef_hbm, ex_hbm, src_hbm, dst_hbm, ag_hbm,
             di0, di1, si0, si1, vb0, vb1, eb0, eb1, xb0, xb1, dsc,
             sem0, sem1, isem0, isem1, agg_sp):
        cid = lax.axis_index("c")
        sid = lax.axis_index("s")
        wid = _worker_id()
        lane = lax.iota(jnp.int32, L)
        nmy = jnp.where(wid < nch4 - (hi4 - 1) * NW, hi4, hi4 - 1)
        slots = ((di0, si0, vb0, eb0, xb0, sem0, isem0),
                 (di1, si1, vb1, eb1, xb1, sem1, isem1))

        @pl.loop(0, C4)
        def _(r):
            @pl.loop(0, D, step=L)
            def _(c0):
                vb0[r, pl.ds(c0, L)] = jnp.zeros((L,), jnp.float32)

        RT = 624
        row0 = sid * RT

        @pl.when(sid < 15)
        def _():
            for b in range(RT // C4):
                pltpu.sync_copy(vb0, agg_sp.at[pl.ds(row0 + b * C4, C4)])
            pltpu.sync_copy(vb0.at[pl.ds(0, RT - (RT // C4) * C4)],
                            agg_sp.at[pl.ds(row0 + (RT // C4) * C4,
                                            RT - (RT // C4) * C4)])

        @pl.when(sid == 15)
        def _():
            for b in range((N - 15 * RT) // C4):
                pltpu.sync_copy(vb0, agg_sp.at[pl.ds(row0 + b * C4, C4)])

        plsc.subcore_barrier()

        def issue_idx(s, t):
            di, si, _, _, _, _, isem = slots[s]
            off = (wid + NW * t) * C4
            pltpu.async_copy(dst_hbm.at[pl.ds(off, C4)], di.at[0], isem)
            pltpu.async_copy(src_hbm.at[pl.ds(off, C4)], si.at[0], isem)

        def issue_fetch(s, t):
            di, si, vb, eb, xb, gsem, isem = slots[s]
            off = (wid + NW * t) * C4
            pltpu.make_async_copy(dst_hbm.at[pl.ds(off, C4)], di.at[0],
                                  isem).wait()
            pltpu.make_async_copy(src_hbm.at[pl.ds(off, C4)], si.at[0],
                                  isem).wait()
            pltpu.async_copy(v_hbm.at[si.at[0]], vb, gsem)
            pltpu.async_copy(ef_hbm.at[pl.ds(off, C4)], eb, gsem)
            pltpu.async_copy(ex_hbm.at[pl.ds(off, C4)], xb, gsem)

        def wait_fetch(s, t):
            di, si, vb, eb, xb, gsem, isem = slots[s]
            off = (wid + NW * t) * C4
            pltpu.make_async_copy(v_hbm.at[si.at[0]], vb, gsem).wait()
            pltpu.make_async_copy(ef_hbm.at[pl.ds(off, C4)], eb,
                                  gsem).wait()
            pltpu.make_async_copy(ex_hbm.at[pl.ds(off, C4)], xb,
                                  gsem).wait()
            for j in range(C4 // L):
                dsc[0, pl.ds(L * j, L)] = di[0, pl.ds(L * j, L)]

        def compute_scatter(s, t):
            di, si, vb, eb, xb, gsem, isem = slots[s]
            for g in range(C4 // L):
                erow = lane + L * g
                exs = [plsc.load_gather(xb, [erow, lane * 0 + h])
                       for h in range(H)]

                @pl.loop(0, L)
                def _(e16):
                    row = L * g + e16
                    sel = lane * 0 + e16
                    bc = [_vshuffle(exs[h], sel) for h in range(H)]
                    for j in range(D // L):
                        vb[row, pl.ds(L * j, L)] = (
                            (vb[row, pl.ds(L * j, L)]
                             + eb[row, pl.ds(L * j, L)]) * bc[j // 2])

            pltpu.sync_copy(vb, agg_sp.at[dsc.at[0]], add=True)

        @pl.when(0 < nmy)
        def _():
            issue_idx(0, 0)

        @pl.when(1 < nmy)
        def _():
            issue_idx(1, 1)

        @pl.when(0 < nmy)
        def _():
            issue_fetch(0, 0)

        @pl.loop(0, (hi4 + 1) // 2)
        def _(i):
            t0 = 2 * i
            t1 = 2 * i + 1

            @pl.when(t1 < nmy)
            def _():
                issue_fetch(1, t1)

            @pl.when(t0 < nmy)
            def _():
                wait_fetch(0, t0)

            @pl.when(t0 + 2 < nmy)
            def _():
                issue_idx(0, t0 + 2)

            @pl.when(t0 < nmy)
            def _():
                compute_scatter(0, t0)

            @pl.when(t1 + 1 < nmy)
            def _():
                issue_fetch(0, t1 + 1)

            @pl.when(t1 < nmy)
            def _():
                wait_fetch(1, t1)

            @pl.when(t1 + 2 < nmy)
            def _():
                issue_idx(1, t1 + 2)

            @pl.when(t1 < nmy)
            def _():
                compute_scatter(1, t1)

        plsc.subcore_barrier()

        @pl.when(sid < 15)
        def _():
            for b in range(RT // C):
                pltpu.sync_copy(agg_sp.at[pl.ds(row0 + b * C, C)],
                                ag_hbm.at[cid, pl.ds(row0 + b * C, C)])
            rem = RT - (RT // C) * C
            pltpu.sync_copy(
                agg_sp.at[pl.ds(row0 + (RT // C) * C, rem)],
                ag_hbm.at[cid, pl.ds(row0 + (RT // C) * C, rem)])

        @pl.when(sid == 15)
        def _():
            nv = N - 15 * RT
            for b in range(nv // C):
                pltpu.sync_copy(agg_sp.at[pl.ds(row0 + b * C, C)],
                                ag_hbm.at[cid, pl.ds(row0 + b * C, C)])

    return kern(v, ef, ex, src, dst)


def kernel(x, edge_index, edge_attr, W_node, W_edge, Wq, Wk, Wv):
    src = edge_index[0]
    dst = edge_index[1]
    xt, q, k, v = _tc_tables(x, W_node, Wq, Wk, Wv)
    ef = _tc_edge_features(edge_attr, W_edge)
    lo = _sc_logits(q, k, ef, src, dst)
    m_parts = _sc_segmax(lo, dst)
    m_final = _sc_combine(m_parts, True)
    ex, den_parts = _sc_exp_den(lo, dst, m_final)
    den_final = _sc_combine(den_parts, False)
    agg_part = _sc_aggregate(v, ef, ex, src, dst)
    den2d = den_final[:N * H].reshape(N, H)
    return _tc_final(xt, agg_part, den2d)


# speedup vs baseline: 3.9832x; 1.0093x over previous
import dataclasses
import functools

import jax
import jax.numpy as jnp
from jax import lax
from jax.experimental import pallas as pl
from jax.experimental.pallas import tpu as pltpu
from jax.experimental.pallas import tpu_sc as plsc

N = 10000
E = 320000
DF = 128
DE = 16
D = 128
H = 4
DH = D // H
L = 16
NW = 32
C = 128
NCHUNKS = E // C
NCH_HI = -(-NCHUNKS // NW)
NP = 10240
NPH = NP * H
SLICE = NPH // 16
AGG_ROWS = NP // 16
INV_SQRT_DH = 1.0 / (DH ** 0.5)
NEG_INF = float("-inf")


def _mesh():
    return plsc.VectorSubcoreMesh(core_axis_name="c", subcore_axis_name="s")


def _sc_params():
    cp = pltpu.CompilerParams()
    if "needs_layout_passes" in pltpu.CompilerParams.__dataclass_fields__:
        cp = dataclasses.replace(cp, needs_layout_passes=False)
    return cp


def _worker_id():
    return lax.axis_index("s") * 2 + lax.axis_index("c")


def _vshuffle(x, idx):
    dn = lax.GatherDimensionNumbers(
        offset_dims=(), collapsed_slice_dims=(0,), start_index_map=(0,))
    return lax.gather(x, idx[:, None], dn, (1,),
                      mode=lax.GatherScatterMode.PROMISE_IN_BOUNDS)


def _dedup_combine(dstv, vals, is_max):
    lane = lax.iota(jnp.int32, L)
    sdst, perm = plsc.sort_key_val(dstv, lane)
    steps = []
    for d in (1, 2, 4, 8):
        nb = jnp.minimum(lane + d, L - 1)
        same = (_vshuffle(sdst, nb) == sdst) & (lane < L - d)
        steps.append((nb, same))
    prev = jnp.maximum(lane - 1, 0)
    firstm = (lane == 0) | (_vshuffle(sdst, prev) != sdst)
    ident = NEG_INF if is_max else 0.0
    comb = jnp.maximum if is_max else jnp.add
    out = []
    for v in vals:
        sv = _vshuffle(v, perm)
        for nb, same in steps:
            sv = comb(sv, jnp.where(same, _vshuffle(sv, nb), ident))
        out.append(sv)
    return sdst, firstm, out


def _foreach_chunk(wid, fn):
    @pl.loop(0, NCH_HI)
    def _(t):
        cidx = wid + NW * t

        @pl.when(cidx < NCHUNKS)
        def _():
            fn(cidx * C)


def _tc_tables(x, w_node, wq, wk, wv):
    blk = 1000

    def body(x_ref, wn_ref, wq_ref, wk_ref, wv_ref,
             xt_ref, q_ref, k_ref, v_ref):
        xt = jnp.dot(x_ref[...], wn_ref[...],
                     preferred_element_type=jnp.float32)
        xt_ref[...] = xt
        q_ref[...] = jnp.dot(xt, wq_ref[...],
                             preferred_element_type=jnp.float32)
        k_ref[...] = jnp.dot(xt, wk_ref[...],
                             preferred_element_type=jnp.float32)
        v_ref[...] = jnp.dot(xt, wv_ref[...],
                             preferred_element_type=jnp.float32)

    w_spec = pl.BlockSpec((DF, D), lambda i: (0, 0))
    r_spec = pl.BlockSpec((blk, D), lambda i: (i, 0))
    shp = jax.ShapeDtypeStruct((N, D), jnp.float32)
    return pl.pallas_call(
        body, grid=(N // blk,),
        in_specs=[pl.BlockSpec((blk, DF), lambda i: (i, 0)),
                  w_spec, w_spec, w_spec, w_spec],
        out_specs=[r_spec, r_spec, r_spec, r_spec],
        out_shape=[shp, shp, shp, shp],
    )(x, w_node, wq, wk, wv)


def _tc_edge_features(edge_attr, w_edge):
    blk = 4000

    def body(ea_ref, we_ref, ef_ref):
        ef_ref[...] = jnp.dot(ea_ref[...], we_ref[...],
                              preferred_element_type=jnp.float32)

    return pl.pallas_call(
        body, grid=(E // blk,),
        in_specs=[pl.BlockSpec((blk, DE), lambda i: (i, 0)),
                  pl.BlockSpec((DE, D), lambda i: (0, 0))],
        out_specs=pl.BlockSpec((blk, D), lambda i: (i, 0)),
        out_shape=jax.ShapeDtypeStruct((E, D), jnp.float32),
    )(edge_attr, w_edge)


def _tc_final(xt, agg_part, den3d):
    blk = 1000

    def body(xt_ref, ag_ref, dn_ref, o_ref):
        den = jnp.sum(dn_ref[...], axis=0)
        den128 = jnp.repeat(den, DH, axis=1)
        agg = (ag_ref[0] + ag_ref[1]) / (den128 + 1e-9)
        o_ref[...] = jnp.maximum(xt_ref[...] + agg, 0.0)

    return pl.pallas_call(
        body, grid=(N // blk,),
        in_specs=[pl.BlockSpec((blk, D), lambda i: (i, 0)),
                  pl.BlockSpec((2, blk, D), lambda i: (0, i, 0)),
                  pl.BlockSpec((NW, blk, H), lambda i: (0, i, 0))],
        out_specs=pl.BlockSpec((blk, D), lambda i: (i, 0)),
        out_shape=jax.ShapeDtypeStruct((N, D), jnp.float32),
    )(xt, agg_part, den3d)


def _sc_logits(q, k, ef, src, dst):
    @functools.partial(
        pl.kernel,
        out_type=jax.ShapeDtypeStruct((H, E), jnp.float32),
        mesh=_mesh(),
        compiler_params=_sc_params(),
        scratch_types=[
            pltpu.VMEM((1, C), jnp.int32),
            pltpu.VMEM((1, C), jnp.int32),
            pltpu.VMEM((1, C), jnp.int32),
            pltpu.VMEM((1, C), jnp.int32),
            pltpu.VMEM((C, D), jnp.float32),
            pltpu.VMEM((C, D), jnp.float32),
            pltpu.VMEM((C, D), jnp.float32),
            pltpu.VMEM((C, D), jnp.float32),
            pltpu.VMEM((C, D), jnp.float32),
            pltpu.VMEM((C, D), jnp.float32),
            pltpu.VMEM((H, C), jnp.float32),
            pltpu.VMEM((H, C), jnp.float32),
            pltpu.SemaphoreType.DMA,
            pltpu.SemaphoreType.DMA,
            pltpu.SemaphoreType.DMA,
            pltpu.SemaphoreType.DMA,
            pltpu.SemaphoreType.DMA,
            pltpu.SemaphoreType.DMA,
        ],
    )
    def kern(q_hbm, k_hbm, ef_hbm, src_hbm, dst_hbm, lo_hbm,
             di0, di1, si0, si1, qb0, qb1, kb0, kb1, eb0, eb1, lb0, lb1,
             gsem0, gsem1, isem0, isem1, lsem0, lsem1):
        wid = _worker_id()
        nmy = jnp.where(wid < NCHUNKS - (NCH_HI - 1) * NW,
                        NCH_HI, NCH_HI - 1)
        lane = lax.iota(jnp.int32, L)
        lastm = lane == (L - 1)
        slots = ((di0, si0, qb0, kb0, eb0, lb0, gsem0, isem0, lsem0),
                 (di1, si1, qb1, kb1, eb1, lb1, gsem1, isem1, lsem1))

        def issue_idx(s, t):
            di, si, _, _, _, _, _, isem, _ = slots[s]
            off = (wid + NW * t) * C
            pltpu.async_copy(dst_hbm.at[pl.ds(off, C)], di.at[0], isem)
            pltpu.async_copy(src_hbm.at[pl.ds(off, C)], si.at[0], isem)

        def issue_gather(s, t):
            di, si, qb, kb, eb, _, gsem, isem, _ = slots[s]
            off = (wid + NW * t) * C
            pltpu.make_async_copy(dst_hbm.at[pl.ds(off, C)], di.at[0],
                                  isem).wait()
            pltpu.make_async_copy(src_hbm.at[pl.ds(off, C)], si.at[0],
                                  isem).wait()
            pltpu.async_copy(q_hbm.at[di.at[0]], qb, gsem)
            pltpu.async_copy(k_hbm.at[si.at[0]], kb, gsem)
            pltpu.async_copy(ef_hbm.at[pl.ds(off, C)], eb, gsem)

        def wait_gathers(s, t):
            di, si, qb, kb, eb, _, gsem, _, _ = slots[s]
            off = (wid + NW * t) * C
            pltpu.make_async_copy(q_hbm.at[di.at[0]], qb, gsem).wait()
            pltpu.make_async_copy(k_hbm.at[si.at[0]], kb, gsem).wait()
            pltpu.make_async_copy(ef_hbm.at[pl.ds(off, C)], eb,
                                  gsem).wait()

        def compute(s, t):
            _, _, qb, kb, eb, lb, _, _, lsem = slots[s]
            off = (wid + NW * t) * C

            @pl.when(t >= 2)
            def _():
                pltpu.make_async_copy(lb, lo_hbm.at[:, pl.ds(off, C)],
                                      lsem).wait()

            @pl.loop(0, C)
            def _(e):
                he = lane * 0 + e
                for h in range(H):
                    j0, j1 = 2 * h, 2 * h + 1
                    t_ = (qb[e, pl.ds(L * j0, L)]
                          * (kb[e, pl.ds(L * j0, L)]
                             + eb[e, pl.ds(L * j0, L)])
                          + qb[e, pl.ds(L * j1, L)]
                          * (kb[e, pl.ds(L * j1, L)]
                             + eb[e, pl.ds(L * j1, L)]))
                    cs = plsc.cumsum(t_) * INV_SQRT_DH
                    plsc.store_scatter(lb, [lane * 0 + h, he], cs,
                                       mask=lastm)

            pltpu.async_copy(lb, lo_hbm.at[:, pl.ds(off, C)], lsem)

        @pl.when(0 < nmy)
        def _():
            issue_idx(0, 0)

        @pl.when(1 < nmy)
        def _():
            issue_idx(1, 1)

        @pl.when(0 < nmy)
        def _():
            issue_gather(0, 0)

        @pl.loop(0, (NCH_HI + 1) // 2)
        def _(i):
            t0 = 2 * i
            t1 = 2 * i + 1

            @pl.when(t1 < nmy)
            def _():
                issue_gather(1, t1)

            @pl.when(t0 < nmy)
            def _():
                wait_gathers(0, t0)

            @pl.when(t0 + 2 < nmy)
            def _():
                issue_idx(0, t0 + 2)

            @pl.when(t0 < nmy)
            def _():
                compute(0, t0)

            @pl.when(t1 + 1 < nmy)
            def _():
                issue_gather(0, t1 + 1)

            @pl.when(t1 < nmy)
            def _():
                wait_gathers(1, t1)

            @pl.when(t1 + 2 < nmy)
            def _():
                issue_idx(1, t1 + 2)

            @pl.when(t1 < nmy)
            def _():
                compute(1, t1)

        @pl.when(nmy >= 2)
        def _():
            pltpu.make_async_copy(lb0, lo_hbm.at[:, pl.ds(0, C)],
                                  lsem0).wait()
            pltpu.make_async_copy(lb1, lo_hbm.at[:, pl.ds(0, C)],
                                  lsem1).wait()

    return kern(q, k, ef, src, dst)


def _sc_segmax(lo, dst):
    @functools.partial(
        pl.kernel,
        out_type=jax.ShapeDtypeStruct((NW * NPH,), jnp.float32),
        mesh=_mesh(),
        compiler_params=_sc_params(),
        scratch_types=[
            pltpu.VMEM((NPH,), jnp.float32),
            pltpu.VMEM((H, C), jnp.float32),
            pltpu.VMEM((H, C), jnp.float32),
            pltpu.VMEM((1, C), jnp.int32),
            pltpu.VMEM((1, C), jnp.int32),
            pltpu.SemaphoreType.DMA,
            pltpu.SemaphoreType.DMA,
        ],
    )
    def kern(lo_hbm, dst_hbm, mp_hbm, m_loc, lb0, lb1, di0, di1,
             gsem0, gsem1):
        wid = _worker_id()
        nmy = jnp.where(wid < NCHUNKS - (NCH_HI - 1) * NW,
                        NCH_HI, NCH_HI - 1)
        slots = ((di0, lb0, gsem0), (di1, lb1, gsem1))

        @pl.loop(0, NPH, step=L)
        def _(i):
            m_loc[pl.ds(i, L)] = jnp.full((L,), NEG_INF, jnp.float32)

        def issue(s, t):
            di, lb, gsem = slots[s]
            off = (wid + NW * t) * C
            pltpu.async_copy(dst_hbm.at[pl.ds(off, C)], di.at[0], gsem)
            pltpu.async_copy(lo_hbm.at[:, pl.ds(off, C)], lb, gsem)

        def consume(s, t):
            di, lb, gsem = slots[s]
            off = (wid + NW * t) * C
            pltpu.make_async_copy(dst_hbm.at[pl.ds(off, C)], di.at[0],
                                  gsem).wait()
            pltpu.make_async_copy(lo_hbm.at[:, pl.ds(off, C)], lb,
                                  gsem).wait()
            for g in range(C // L):
                dstv = di[0, pl.ds(L * g, L)]
                vals = [lb[h, pl.ds(L * g, L)] for h in range(H)]
                sdst, firstm, svals = _dedup_combine(dstv, vals, True)
                b4 = sdst * H
                for h in range(H):
                    idx = b4 + h
                    cur = plsc.load_gather(m_loc, [idx])
                    plsc.store_scatter(m_loc, [idx],
                                       jnp.maximum(cur, svals[h]),
                                       mask=firstm)

        @pl.when(0 < nmy)
        def _():
            issue(0, 0)

        @pl.loop(0, (NCH_HI + 1) // 2)
        def _(i):
            t0 = 2 * i
            t1 = 2 * i + 1

            @pl.when(t1 < nmy)
            def _():
                issue(1, t1)

            @pl.when(t0 < nmy)
            def _():
                consume(0, t0)

            @pl.when(t1 + 1 < nmy)
            def _():
                issue(0, t1 + 1)

            @pl.when(t1 < nmy)
            def _():
                consume(1, t1)

        pltpu.sync_copy(m_loc, mp_hbm.at[pl.ds(wid * NPH, NPH)])

    return kern(lo, dst)


def _tc_combine_max(parts):
    blk = NPH // 8

    def body(p_ref, o_ref):
        o_ref[...] = jnp.max(p_ref[...], axis=0)

    return pl.pallas_call(
        body, grid=(NPH // blk,),
        in_specs=[pl.BlockSpec((NW, blk), lambda i: (0, i))],
        out_specs=pl.BlockSpec((blk,), lambda i: (i,)),
        out_shape=jax.ShapeDtypeStruct((NPH,), jnp.float32),
    )(parts.reshape(NW, NPH))


def _sc_exp_den(lo, dst, m_final):
    @functools.partial(
        pl.kernel,
        out_type=(jax.ShapeDtypeStruct((E, H), jnp.float32),
                  jax.ShapeDtypeStruct((NW * NPH,), jnp.float32)),
        mesh=_mesh(),
        compiler_params=_sc_params(),
        scratch_types=[
            pltpu.VMEM((NPH,), jnp.float32),
            pltpu.VMEM((NPH,), jnp.float32),
            pltpu.VMEM((H, C), jnp.float32),
            pltpu.VMEM((H, C), jnp.float32),
            pltpu.VMEM((C, H), jnp.float32),
            pltpu.VMEM((C, H), jnp.float32),
            pltpu.VMEM((1, C), jnp.int32),
            pltpu.VMEM((1, C), jnp.int32),
            pltpu.SemaphoreType.DMA,
            pltpu.SemaphoreType.DMA,
            pltpu.SemaphoreType.DMA,
            pltpu.SemaphoreType.DMA,
        ],
    )
    def kern(lo_hbm, dst_hbm, m_hbm, ex_hbm, dp_hbm,
             m_loc, den_loc, lb0, lb1, exb0, exb1, di0, di1,
             gsem0, gsem1, xsem0, xsem1):
        wid = _worker_id()
        lane = lax.iota(jnp.int32, L)
        nmy = jnp.where(wid < NCHUNKS - (NCH_HI - 1) * NW,
                        NCH_HI, NCH_HI - 1)
        slots = ((di0, lb0, exb0, gsem0, xsem0),
                 (di1, lb1, exb1, gsem1, xsem1))

        pltpu.sync_copy(m_hbm, m_loc)

        @pl.loop(0, NPH, step=L)
        def _(i):
            den_loc[pl.ds(i, L)] = jnp.zeros((L,), jnp.float32)

        def issue(s, t):
            di, lb, exb, gsem, xsem = slots[s]
            off = (wid + NW * t) * C
            pltpu.async_copy(dst_hbm.at[pl.ds(off, C)], di.at[0], gsem)
            pltpu.async_copy(lo_hbm.at[:, pl.ds(off, C)], lb, gsem)

        def consume(s, t):
            di, lb, exb, gsem, xsem = slots[s]
            off = (wid + NW * t) * C
            pltpu.make_async_copy(dst_hbm.at[pl.ds(off, C)], di.at[0],
                                  gsem).wait()
            pltpu.make_async_copy(lo_hbm.at[:, pl.ds(off, C)], lb,
                                  gsem).wait()

            @pl.when(t >= 2)
            def _():
                pltpu.make_async_copy(exb, ex_hbm.at[pl.ds(off, C)],
                                      xsem).wait()

            for g in range(C // L):
                dstv = di[0, pl.ds(L * g, L)]
                b4 = dstv * H
                erow = lane + L * g
                exs = []
                for h in range(H):
                    mg = plsc.load_gather(m_loc, [b4 + h])
                    exv = jnp.exp(lb[h, pl.ds(L * g, L)] - mg)
                    plsc.store_scatter(exb, [erow, lane * 0 + h], exv)
                    exs.append(exv)
                sdst, firstm, svals = _dedup_combine(dstv, exs, False)
                sb4 = sdst * H
                for h in range(H):
                    idx = sb4 + h
                    cur = plsc.load_gather(den_loc, [idx])
                    plsc.store_scatter(den_loc, [idx], cur + svals[h],
                                       mask=firstm)

            pltpu.async_copy(exb, ex_hbm.at[pl.ds(off, C)], xsem)

        @pl.when(0 < nmy)
        def _():
            issue(0, 0)

        @pl.loop(0, (NCH_HI + 1) // 2)
        def _(i):
            t0 = 2 * i
            t1 = 2 * i + 1

            @pl.when(t1 < nmy)
            def _():
                issue(1, t1)

            @pl.when(t0 < nmy)
            def _():
                consume(0, t0)

            @pl.when(t1 + 1 < nmy)
            def _():
                issue(0, t1 + 1)

            @pl.when(t1 < nmy)
            def _():
                consume(1, t1)

        @pl.when(nmy >= 2)
        def _():
            pltpu.make_async_copy(exb0, ex_hbm.at[pl.ds(0, C)],
                                  xsem0).wait()
            pltpu.make_async_copy(exb1, ex_hbm.at[pl.ds(0, C)],
                                  xsem1).wait()

        pltpu.sync_copy(den_loc, dp_hbm.at[pl.ds(wid * NPH, NPH)])

    return kern(lo, dst, m_final)


def _sc_aggregate(v, ef, ex, src, dst):
    C4 = 64
    nch4 = E // C4
    hi4 = -(-nch4 // NW)

    @functools.partial(
        pl.kernel,
        out_type=jax.ShapeDtypeStruct((2, N, D), jnp.float32),
        mesh=_mesh(),
        compiler_params=_sc_params(),
        scratch_types=[
            pltpu.VMEM((1, C4), jnp.int32),
            pltpu.VMEM((1, C4), jnp.int32),
            pltpu.VMEM((1, C4), jnp.int32),
            pltpu.VMEM((1, C4), jnp.int32),
            pltpu.VMEM((C4, D), jnp.float32),
            pltpu.VMEM((C4, D), jnp.float32),
            pltpu.VMEM((C4, D), jnp.float32),
            pltpu.VMEM((C4, D), jnp.float32),
            pltpu.VMEM((C4, H), jnp.float32),
            pltpu.VMEM((C4, H), jnp.float32),
            pltpu.VMEM((1, C4), jnp.int32),
            pltpu.SemaphoreType.DMA,
            pltpu.SemaphoreType.DMA,
            pltpu.SemaphoreType.DMA,
            pltpu.SemaphoreType.DMA,
            pltpu.VMEM_SHARED((N, D), jnp.float32),
        ],
    )
    def kern(v_hbm, ef_hbm, ex_hbm, src_hbm, dst_hbm, ag_hbm,
             di0, di1, si0, si1, vb0, vb1, eb0, eb1, xb0, xb1, dsc,
             sem0, sem1, isem0, isem1, agg_sp):
        cid = lax.axis_index("c")
        sid = lax.axis_index("s")
        wid = _worker_id()
        lane = lax.iota(jnp.int32, L)
        nmy = jnp.where(wid < nch4 - (hi4 - 1) * NW, hi4, hi4 - 1)
        slots = ((di0, si0, vb0, eb0, xb0, sem0, isem0),
                 (di1, si1, vb1, eb1, xb1, sem1, isem1))

        @pl.loop(0, C4)
        def _(r):
            @pl.loop(0, D, step=L)
            def _(c0):
                vb0[r, pl.ds(c0, L)] = jnp.zeros((L,), jnp.float32)

        RT = 624
        row0 = sid * RT

        @pl.when(sid < 15)
        def _():
            for b in range(RT // C4):
                pltpu.sync_copy(vb0, agg_sp.at[pl.ds(row0 + b * C4, C4)])
            pltpu.sync_copy(vb0.at[pl.ds(0, RT - (RT // C4) * C4)],
                            agg_sp.at[pl.ds(row0 + (RT // C4) * C4,
                                            RT - (RT // C4) * C4)])

        @pl.when(sid == 15)
        def _():
            for b in range((N - 15 * RT) // C4):
                pltpu.sync_copy(vb0, agg_sp.at[pl.ds(row0 + b * C4, C4)])

        plsc.subcore_barrier()

        def issue_idx(s, t):
            di, si, _, _, _, _, isem = slots[s]
            off = (wid + NW * t) * C4
            pltpu.async_copy(dst_hbm.at[pl.ds(off, C4)], di.at[0], isem)
            pltpu.async_copy(src_hbm.at[pl.ds(off, C4)], si.at[0], isem)

        def issue_fetch(s, t):
            di, si, vb, eb, xb, gsem, isem = slots[s]
            off = (wid + NW * t) * C4
            pltpu.make_async_copy(dst_hbm.at[pl.ds(off, C4)], di.at[0],
                                  isem).wait()
            pltpu.make_async_copy(src_hbm.at[pl.ds(off, C4)], si.at[0],
                                  isem).wait()
            pltpu.async_copy(v_hbm.at[si.at[0]], vb, gsem)
            pltpu.async_copy(ef_hbm.at[pl.ds(off, C4)], eb, gsem)
            pltpu.async_copy(ex_hbm.at[pl.ds(off, C4)], xb, gsem)

        def wait_fetch(s, t):
            di, si, vb, eb, xb, gsem, isem = slots[s]
            off = (wid + NW * t) * C4
            pltpu.make_async_copy(v_hbm.at[si.at[0]], vb, gsem).wait()
            pltpu.make_async_copy(ef_hbm.at[pl.ds(off, C4)], eb,
                                  gsem).wait()
            pltpu.make_async_copy(ex_hbm.at[pl.ds(off, C4)], xb,
                                  gsem).wait()
            for j in range(C4 // L):
                dsc[0, pl.ds(L * j, L)] = di[0, pl.ds(L * j, L)]

        def compute_scatter(s, t):
            di, si, vb, eb, xb, gsem, isem = slots[s]
            for g in range(C4 // L):
                erow = lane + L * g
                exs = [plsc.load_gather(xb, [erow, lane * 0 + h])
                       for h in range(H)]

                @pl.loop(0, L)
                def _(e16):
                    row = L * g + e16
                    sel = lane * 0 + e16
                    bc = [_vshuffle(exs[h], sel) for h in range(H)]
                    for j in range(D // L):
                        vb[row, pl.ds(L * j, L)] = (
                            (vb[row, pl.ds(L * j, L)]
                             + eb[row, pl.ds(L * j, L)]) * bc[j // 2])

            pltpu.sync_copy(vb, agg_sp.at[dsc.at[0]], add=True)

        @pl.when(0 < nmy)
        def _():
            issue_idx(0, 0)

        @pl.when(1 < nmy)
        def _():
            issue_idx(1, 1)

        @pl.when(0 < nmy)
        def _():
            issue_fetch(0, 0)

        @pl.loop(0, (hi4 + 1) // 2)
        def _(i):
            t0 = 2 * i
            t1 = 2 * i + 1

            @pl.when(t1 < nmy)
            def _():
                issue_fetch(1, t1)

            @pl.when(t0 < nmy)
            def _():
                wait_fetch(0, t0)

            @pl.when(t0 + 2 < nmy)
            def _():
                issue_idx(0, t0 + 2)

            @pl.when(t0 < nmy)
            def _():
                compute_scatter(0, t0)

            @pl.when(t1 + 1 < nmy)
            def _():
                issue_fetch(0, t1 + 1)

            @pl.when(t1 < nmy)
            def _():
                wait_fetch(1, t1)

            @pl.when(t1 + 2 < nmy)
            def _():
                issue_idx(1, t1 + 2)

            @pl.when(t1 < nmy)
            def _():
                compute_scatter(1, t1)

        plsc.subcore_barrier()

        @pl.when(sid < 15)
        def _():
            for b in range(RT // C):
                pltpu.sync_copy(agg_sp.at[pl.ds(row0 + b * C, C)],
                                ag_hbm.at[cid, pl.ds(row0 + b * C, C)])
            rem = RT - (RT // C) * C
            pltpu.sync_copy(
                agg_sp.at[pl.ds(row0 + (RT // C) * C, rem)],
                ag_hbm.at[cid, pl.ds(row0 + (RT // C) * C, rem)])

        @pl.when(sid == 15)
        def _():
            nv = N - 15 * RT
            for b in range(nv // C):
                pltpu.sync_copy(agg_sp.at[pl.ds(row0 + b * C, C)],
                                ag_hbm.at[cid, pl.ds(row0 + b * C, C)])

    return kern(v, ef, ex, src, dst)


def kernel(x, edge_index, edge_attr, W_node, W_edge, Wq, Wk, Wv):
    src = edge_index[0]
    dst = edge_index[1]
    xt, q, k, v = _tc_tables(x, W_node, Wq, Wk, Wv)
    ef = _tc_edge_features(edge_attr, W_edge)
    lo = _sc_logits(q, k, ef, src, dst)
    m_parts = _sc_segmax(lo, dst)
    m_final = _tc_combine_max(m_parts)
    ex, den_parts = _sc_exp_den(lo, dst, m_final)
    agg_part = _sc_aggregate(v, ef, ex, src, dst)
    den3d = den_parts.reshape(NW, NP, H)[:, :N]
    return _tc_final(xt, agg_part, den3d)
